# Initial kernel scaffold; baseline (speedup 1.0000x reference)
import jax
import jax.numpy as jnp
from jax.experimental import pallas as pl


def kernel(x, edge_index, edge_attr, params):
    raise NotImplementedError("write your pallas kernel here")


# speedup vs baseline: 1.6979x; 1.6979x over previous
import functools

import jax
import jax.numpy as jnp
from jax import lax
from jax.experimental import pallas as pl
from jax.experimental.pallas import tpu as pltpu
from jax.experimental.pallas import tpu_sc as plsc

N = 10000
NPAD = 10240
E = 160000
HID = 512
DC = 128
NCH = HID // DC
NS = 16
EPT = E // NS
BE = 80
NBLK = EPT // BE
ROWS_PT = NPAD // NS
ZROWS = 128
BN = 1024
BEDGE = 2000

_sc_mesh = plsc.VectorSubcoreMesh(core_axis_name="c", subcore_axis_name="s")

_PASSES = (
    (0, 0, 0), (0, 1, 0), (1, 0, 0), (1, 1, 0),
    (0, 2, 1), (0, 3, 1), (1, 2, 1), (1, 3, 1),
)


def _mlp2_body(x_ref, w1_ref, b1_ref, w2_ref, b2_ref, *o_refs):
    q = jnp.maximum(
        jnp.dot(x_ref[...], w1_ref[...], preferred_element_type=jnp.float32)
        + b1_ref[...], 0.0)
    hh = jnp.dot(q, w2_ref[...], preferred_element_type=jnp.float32) + b2_ref[...]
    for c in range(NCH):
        o_refs[c][...] = hh[:, c * DC:(c + 1) * DC]


def _node_mlp(h, w1t, b1, w2t, b2):
    din = h.shape[1]
    return pl.pallas_call(
        _mlp2_body,
        grid=(NPAD // BN,),
        in_specs=[
            pl.BlockSpec((BN, din), lambda i: (i, 0)),
            pl.BlockSpec((din, HID), lambda i: (0, 0)),
            pl.BlockSpec((1, HID), lambda i: (0, 0)),
            pl.BlockSpec((HID, HID), lambda i: (0, 0)),
            pl.BlockSpec((1, HID), lambda i: (0, 0)),
        ],
        out_specs=[pl.BlockSpec((BN, DC), lambda i: (i, 0)) for _ in range(NCH)],
        out_shape=[jax.ShapeDtypeStruct((NPAD, DC), jnp.float32) for _ in range(NCH)],
    )(h, w1t, b1, w2t, b2)


def _edge_relu_body(a_ref, w_ref, b_ref, *o_refs):
    q = jnp.maximum(
        jnp.dot(a_ref[...], w_ref[...], preferred_element_type=jnp.float32)
        + b_ref[...], 0.0)
    for c in range(NCH):
        o_refs[c][...] = q[:, c * DC:(c + 1) * DC]


def _edge_mlp(ea, w1t, b1):
    ed = ea.shape[1]
    return pl.pallas_call(
        _edge_relu_body,
        grid=(E // BEDGE,),
        in_specs=[
            pl.BlockSpec((BEDGE, ed), lambda i: (i, 0)),
            pl.BlockSpec((ed, HID), lambda i: (0, 0)),
            pl.BlockSpec((1, HID), lambda i: (0, 0)),
        ],
        out_specs=[pl.BlockSpec((BEDGE, DC), lambda i: (i, 0)) for _ in range(NCH)],
        out_shape=[jax.ShapeDtypeStruct((E, DC), jnp.float32) for _ in range(NCH)],
    )(ea, w1t, b1)


def _fold_body(s10, s11, s12, s13, s20, s21, s22, s23, c0_ref, c1_ref,
               mwt_ref, c2t_ref, dvec_ref, uwt_ref, ub_ref, o_ref):
    s1 = (s10, s11, s12, s13)
    s2 = (s20, s21, s22, s23)
    cnt = c0_ref[:, 0:1] + c1_ref[:, 0:1]
    sums = cnt * dvec_ref[...]
    for c in range(NCH):
        sums += jnp.dot(s1[c][...], mwt_ref[c * DC:(c + 1) * DC, :],
                        preferred_element_type=jnp.float32)
        sums += jnp.dot(s2[c][...], c2t_ref[c * DC:(c + 1) * DC, :],
                        preferred_element_type=jnp.float32)
    inv = 1.0 / jnp.maximum(cnt, 1.0)
    o_ref[...] = (jnp.dot(sums * inv, uwt_ref[...],
                          preferred_element_type=jnp.float32) + ub_ref[...])


def _fold(s1c, s2c, cnt0, cnt1, mwt, c2t, dvec, uwt, ub):
    chunk_spec = [pl.BlockSpec((BN, DC), lambda i: (i, 0)) for _ in range(2 * NCH)]
    return pl.pallas_call(
        _fold_body,
        grid=(NPAD // BN,),
        in_specs=chunk_spec + [
            pl.BlockSpec((BN, DC), lambda i: (i, 0)),
            pl.BlockSpec((BN, DC), lambda i: (i, 0)),
            pl.BlockSpec((HID, HID), lambda i: (0, 0)),
            pl.BlockSpec((HID, HID), lambda i: (0, 0)),
            pl.BlockSpec((1, HID), lambda i: (0, 0)),
            pl.BlockSpec((HID, HID), lambda i: (0, 0)),
            pl.BlockSpec((1, HID), lambda i: (0, 0)),
        ],
        out_specs=pl.BlockSpec((BN, HID), lambda i: (i, 0)),
        out_shape=jax.ShapeDtypeStruct((NPAD, HID), jnp.float32),
    )(*s1c, *s2c, cnt0, cnt1, mwt, c2t, dvec, uwt, ub)


def _linear_body(x_ref, w_ref, b_ref, o_ref):
    o_ref[...] = (jnp.dot(x_ref[...], w_ref[...],
                          preferred_element_type=jnp.float32) + b_ref[...])


def _out_linear(h, wt, b):
    dout = wt.shape[1]
    return pl.pallas_call(
        _linear_body,
        grid=(NPAD // BN,),
        in_specs=[
            pl.BlockSpec((BN, HID), lambda i: (i, 0)),
            pl.BlockSpec((HID, dout), lambda i: (0, 0)),
            pl.BlockSpec((1, dout), lambda i: (0, 0)),
        ],
        out_specs=pl.BlockSpec((BN, dout), lambda i: (i, 0)),
        out_shape=jax.ShapeDtypeStruct((NPAD, dout), jnp.float32),
    )(h, wt, b)


EPT_CNT = E // (2 * NS)
BE_CNT = 40
NBLK_CNT = EPT_CNT // BE_CNT


@functools.partial(
    pl.kernel,
    out_type=(jax.ShapeDtypeStruct((NPAD, DC), jnp.float32),
              jax.ShapeDtypeStruct((NPAD, DC), jnp.float32)),
    mesh=_sc_mesh,
    scratch_types=[
        pltpu.VMEM((BE_CNT,), jnp.int32),
        pltpu.VMEM((BE_CNT, DC), jnp.float32),
        pltpu.VMEM((ZROWS, DC), jnp.float32),
        pltpu.VMEM_SHARED((NPAD, DC), jnp.float32),
    ],
)
def _cnt_kernel(row_hbm, ones_hbm, zer_hbm, out0, out1, rowbuf, onesbuf, zbuf, acc):
    cid = lax.axis_index("c")
    sid = lax.axis_index("s")
    pltpu.sync_copy(ones_hbm, onesbuf)
    pltpu.sync_copy(zer_hbm, zbuf)
    for k in range(ROWS_PT // ZROWS):
        pltpu.sync_copy(zbuf, acc.at[pl.ds(sid * ROWS_PT + k * ZROWS, ZROWS)])
    plsc.subcore_barrier()

    def blk(j, _):
        base = cid * (E // 2) + sid * EPT_CNT + j * BE_CNT
        pltpu.sync_copy(row_hbm.at[pl.ds(base, BE_CNT)], rowbuf)
        pltpu.sync_copy(onesbuf, acc.at[rowbuf], add=True)
        return 0
    lax.fori_loop(0, NBLK_CNT, blk, 0)
    plsc.subcore_barrier()

    @pl.when(cid == 0)
    def _():
        pltpu.sync_copy(acc.at[pl.ds(sid * ROWS_PT, ROWS_PT)],
                        out0.at[pl.ds(sid * ROWS_PT, ROWS_PT)])

    @pl.when(cid == 1)
    def _():
        pltpu.sync_copy(acc.at[pl.ds(sid * ROWS_PT, ROWS_PT)],
                        out1.at[pl.ds(sid * ROWS_PT, ROWS_PT)])


_SEG_OUT = tuple(jax.ShapeDtypeStruct((NPAD, DC), jnp.float32) for _ in range(2 * NCH))


@functools.partial(
    pl.kernel,
    out_type=_SEG_OUT,
    mesh=_sc_mesh,
    scratch_types=[
        pltpu.VMEM((BE,), jnp.int32),
        pltpu.VMEM((BE,), jnp.int32),
        pltpu.VMEM((BE, DC), jnp.float32),
        pltpu.VMEM((ZROWS, DC), jnp.float32),
        pltpu.VMEM_SHARED((NPAD, DC), jnp.float32),
        pltpu.SemaphoreType.DMA,
    ],
)
def _segsum_kernel(hh0, hh1, hh2, hh3, r0, r1, r2, r3, col_hbm, row_hbm, zer_hbm,
                   s10, s11, s12, s13, s20, s21, s22, s23,
                   colbuf, rowbuf, gbuf, zbuf, acc, sem):
    hh = (hh0, hh1, hh2, hh3)
    r = (r0, r1, r2, r3)
    s_out = ((s10, s11, s12, s13), (s20, s21, s22, s23))
    cid = lax.axis_index("c")
    sid = lax.axis_index("s")
    pltpu.sync_copy(zer_hbm, zbuf)

    for src, ch, pc in _PASSES:
        @pl.when(cid == pc)
        def _(src=src, ch=ch):
            for k in range(ROWS_PT // ZROWS):
                pltpu.sync_copy(zbuf, acc.at[pl.ds(sid * ROWS_PT + k * ZROWS, ZROWS)])
            plsc.subcore_barrier()

            def blk(j, _):
                base = sid * EPT + j * BE
                pltpu.sync_copy(row_hbm.at[pl.ds(base, BE)], rowbuf)
                if src == 0:
                    pltpu.sync_copy(col_hbm.at[pl.ds(base, BE)], colbuf)
                    pltpu.async_copy(hh[ch].at[colbuf], gbuf, sem).wait()
                else:
                    pltpu.sync_copy(r[ch].at[pl.ds(base, BE)], gbuf)
                pltpu.sync_copy(gbuf, acc.at[rowbuf], add=True)
                return 0
            lax.fori_loop(0, NBLK, blk, 0)
            plsc.subcore_barrier()
            pltpu.sync_copy(acc.at[pl.ds(sid * ROWS_PT, ROWS_PT)],
                            s_out[src][ch].at[pl.ds(sid * ROWS_PT, ROWS_PT)])


def kernel(x, edge_index, edge_attr, params):
    row = edge_index[0]
    col = edge_index[1]
    h = x.reshape(x.shape[0] * x.shape[1], x.shape[-1])
    h = jnp.pad(h, ((0, NPAD - N), (0, 0)))

    ones_cnt = jnp.ones((BE_CNT, DC), jnp.float32)
    zer_zdc = jnp.zeros((ZROWS, DC), jnp.float32)
    cnt0, cnt1 = _cnt_kernel(row, ones_cnt, zer_zdc)

    for p in params["layers"]:
        hh_c = _node_mlp(h, p["ne_W1"].T, p["ne_b1"].reshape(1, HID),
                         p["ne_W2"].T, p["ne_b2"].reshape(1, HID))
        r_c = _edge_mlp(edge_attr, p["ee_W1"].T, p["ee_b1"].reshape(1, HID))
        s = _segsum_kernel(*hh_c, *r_c, col, row, zer_zdc)
        mwt = p["m_W"].T
        c2t = p["ee_W2"].T @ mwt
        dvec = (p["ee_b2"] @ mwt + p["m_b"]).reshape(1, HID)
        h = _fold(s[:NCH], s[NCH:], cnt0, cnt1, mwt, c2t, dvec,
                  p["u_W"].T, p["u_b"].reshape(1, HID))

    out = _out_linear(h, params["out_W"].T, params["out_b"].reshape(1, -1))
    return out[:N].reshape(x.shape[0], x.shape[1], -1)


# speedup vs baseline: 3.0941x; 1.8223x over previous
import functools

import jax
import jax.numpy as jnp
from jax import lax
from jax.experimental import pallas as pl
from jax.experimental.pallas import tpu as pltpu
from jax.experimental.pallas import tpu_sc as plsc

N = 10000
NPAD = 10240
E = 160000
HID = 512
DC = 128
NCH = HID // DC
NS = 16
EPT = E // NS
BE = 128
ROWS_PT = NPAD // NS
ZROWS = 128
BN = 1024
BEDGE = 2048

_sc_mesh = plsc.VectorSubcoreMesh(core_axis_name="c", subcore_axis_name="s")

_PASSES = (
    (0, 0, 0), (0, 1, 0), (1, 0, 0), (1, 1, 0),
    (0, 2, 1), (0, 3, 1), (1, 2, 1), (1, 3, 1),
)


def _mlp2_body(x_ref, w1_ref, b1_ref, w2_ref, b2_ref, *o_refs):
    q = jnp.maximum(
        jnp.dot(x_ref[...], w1_ref[...], preferred_element_type=jnp.float32)
        + b1_ref[...], 0.0)
    hh = jnp.dot(q, w2_ref[...], preferred_element_type=jnp.float32) + b2_ref[...]
    for c in range(NCH):
        o_refs[c][...] = hh[:, c * DC:(c + 1) * DC]


def _node_mlp(h, w1t, b1, w2t, b2):
    din = h.shape[1]
    return pl.pallas_call(
        _mlp2_body,
        grid=(NPAD // BN,),
        in_specs=[
            pl.BlockSpec((BN, din), lambda i: (i, 0)),
            pl.BlockSpec((din, HID), lambda i: (0, 0)),
            pl.BlockSpec((1, HID), lambda i: (0, 0)),
            pl.BlockSpec((HID, HID), lambda i: (0, 0)),
            pl.BlockSpec((1, HID), lambda i: (0, 0)),
        ],
        out_specs=[pl.BlockSpec((BN, DC), lambda i: (i, 0)) for _ in range(NCH)],
        out_shape=[jax.ShapeDtypeStruct((NPAD, DC), jnp.float32) for _ in range(NCH)],
    )(h, w1t, b1, w2t, b2)


def _edge_relu_body(a_ref, w_ref, b_ref, *o_refs):
    q = jnp.maximum(
        jnp.dot(a_ref[...], w_ref[...], preferred_element_type=jnp.float32)
        + b_ref[...], 0.0)
    for c in range(NCH):
        o_refs[c][...] = q[:, c * DC:(c + 1) * DC]


def _edge_mlp(ea, w1t, b1):
    ed = ea.shape[1]
    ne = ea.shape[0]
    return pl.pallas_call(
        _edge_relu_body,
        grid=(ne // BEDGE,),
        in_specs=[
            pl.BlockSpec((BEDGE, ed), lambda i: (i, 0)),
            pl.BlockSpec((ed, HID), lambda i: (0, 0)),
            pl.BlockSpec((1, HID), lambda i: (0, 0)),
        ],
        out_specs=[pl.BlockSpec((BEDGE, DC), lambda i: (i, 0)) for _ in range(NCH)],
        out_shape=[jax.ShapeDtypeStruct((ne, DC), jnp.float32) for _ in range(NCH)],
    )(ea, w1t, b1)


def _fold_body(s10, s11, s12, s13, s20, s21, s22, s23, c0_ref, c1_ref,
               mwt_ref, c2t_ref, dvec_ref, uwt_ref, ub_ref, o_ref):
    s1 = (s10, s11, s12, s13)
    s2 = (s20, s21, s22, s23)
    cnt = c0_ref[:, 0:1] + c1_ref[:, 0:1]
    sums = cnt * dvec_ref[...]
    for c in range(NCH):
        sums += jnp.dot(s1[c][...], mwt_ref[c * DC:(c + 1) * DC, :],
                        preferred_element_type=jnp.float32)
        sums += jnp.dot(s2[c][...], c2t_ref[c * DC:(c + 1) * DC, :],
                        preferred_element_type=jnp.float32)
    inv = 1.0 / jnp.maximum(cnt, 1.0)
    o_ref[...] = (jnp.dot(sums * inv, uwt_ref[...],
                          preferred_element_type=jnp.float32) + ub_ref[...])


def _fold(s1c, s2c, cnt0, cnt1, mwt, c2t, dvec, uwt, ub):
    chunk_spec = [pl.BlockSpec((BN, DC), lambda i: (i, 0)) for _ in range(2 * NCH)]
    return pl.pallas_call(
        _fold_body,
        grid=(NPAD // BN,),
        in_specs=chunk_spec + [
            pl.BlockSpec((BN, DC), lambda i: (i, 0)),
            pl.BlockSpec((BN, DC), lambda i: (i, 0)),
            pl.BlockSpec((HID, HID), lambda i: (0, 0)),
            pl.BlockSpec((HID, HID), lambda i: (0, 0)),
            pl.BlockSpec((1, HID), lambda i: (0, 0)),
            pl.BlockSpec((HID, HID), lambda i: (0, 0)),
            pl.BlockSpec((1, HID), lambda i: (0, 0)),
        ],
        out_specs=pl.BlockSpec((BN, HID), lambda i: (i, 0)),
        out_shape=jax.ShapeDtypeStruct((NPAD, HID), jnp.float32),
    )(*s1c, *s2c, cnt0, cnt1, mwt, c2t, dvec, uwt, ub)


def _linear_body(x_ref, w_ref, b_ref, o_ref):
    o_ref[...] = (jnp.dot(x_ref[...], w_ref[...],
                          preferred_element_type=jnp.float32) + b_ref[...])


def _out_linear(h, wt, b):
    dout = wt.shape[1]
    return pl.pallas_call(
        _linear_body,
        grid=(NPAD // BN,),
        in_specs=[
            pl.BlockSpec((BN, HID), lambda i: (i, 0)),
            pl.BlockSpec((HID, dout), lambda i: (0, 0)),
            pl.BlockSpec((1, dout), lambda i: (0, 0)),
        ],
        out_specs=pl.BlockSpec((BN, dout), lambda i: (i, 0)),
        out_shape=jax.ShapeDtypeStruct((NPAD, dout), jnp.float32),
    )(h, wt, b)


EPT_CNT = E // (2 * NS)
BE_CNT = 40
NBLK_CNT = EPT_CNT // BE_CNT


@functools.partial(
    pl.kernel,
    out_type=(jax.ShapeDtypeStruct((NPAD, DC), jnp.float32),
              jax.ShapeDtypeStruct((NPAD, DC), jnp.float32)),
    mesh=_sc_mesh,
    scratch_types=[
        pltpu.VMEM((BE_CNT,), jnp.int32),
        pltpu.VMEM((BE_CNT, DC), jnp.float32),
        pltpu.VMEM((ZROWS, DC), jnp.float32),
        pltpu.VMEM_SHARED((NPAD, DC), jnp.float32),
    ],
)
def _cnt_kernel(row_hbm, ones_hbm, zer_hbm, out0, out1, rowbuf, onesbuf, zbuf, acc):
    cid = lax.axis_index("c")
    sid = lax.axis_index("s")
    pltpu.sync_copy(ones_hbm, onesbuf)
    pltpu.sync_copy(zer_hbm, zbuf)
    for k in range(ROWS_PT // ZROWS):
        pltpu.sync_copy(zbuf, acc.at[pl.ds(sid * ROWS_PT + k * ZROWS, ZROWS)])
    plsc.subcore_barrier()

    def blk(j, _):
        base = cid * (E // 2) + sid * EPT_CNT + j * BE_CNT
        pltpu.sync_copy(row_hbm.at[pl.ds(base, BE_CNT)], rowbuf)
        pltpu.sync_copy(onesbuf, acc.at[rowbuf], add=True)
        return 0
    lax.fori_loop(0, NBLK_CNT, blk, 0)
    plsc.subcore_barrier()

    @pl.when(cid == 0)
    def _():
        pltpu.sync_copy(acc.at[pl.ds(sid * ROWS_PT, ROWS_PT)],
                        out0.at[pl.ds(sid * ROWS_PT, ROWS_PT)])

    @pl.when(cid == 1)
    def _():
        pltpu.sync_copy(acc.at[pl.ds(sid * ROWS_PT, ROWS_PT)],
                        out1.at[pl.ds(sid * ROWS_PT, ROWS_PT)])


_SEG_OUT = tuple(jax.ShapeDtypeStruct((NPAD, DC), jnp.float32) for _ in range(2 * NCH))


NBUF = 5
TPE = 10240
NBLK = TPE // BE
GRP = 8
NGRP = NBLK // GRP


@functools.partial(
    pl.kernel,
    out_type=_SEG_OUT,
    mesh=_sc_mesh,
    scratch_types=[
        pltpu.VMEM((2, GRP, BE), jnp.int32),
        pltpu.VMEM((2, GRP, BE), jnp.int32),
        pltpu.VMEM((2, BE, DC), jnp.float32),
        pltpu.VMEM_SHARED((NPAD, DC), jnp.float32),
        pltpu.SemaphoreType.DMA,
        pltpu.SemaphoreType.DMA,
        pltpu.SemaphoreType.DMA,
        pltpu.SemaphoreType.DMA,
        pltpu.SemaphoreType.DMA,
        pltpu.SemaphoreType.DMA,
    ],
)
def _segsum_kernel(hh0, hh1, hh2, hh3, r0, r1, r2, r3, col_hbm, row_hbm, zer_hbm,
                   s10, s11, s12, s13, s20, s21, s22, s23,
                   colring, rowring, gbuf, acc,
                   semg0, semg1, semc0, semc1, semr0, semr1):
    hh = (hh0, hh1, hh2, hh3)
    r = (r0, r1, r2, r3)
    s_out = ((s10, s11, s12, s13), (s20, s21, s22, s23))
    semg = (semg0, semg1)
    semc = (semc0, semc1)
    semr = (semr0, semr1)
    cid = lax.axis_index("c")
    sid = lax.axis_index("s")

    def idx_desc(g, slot):
        return (
            pltpu.make_async_copy(col_hbm.at[sid, pl.ds(g * GRP, GRP)],
                                  colring.at[slot], semc[slot]),
            pltpu.make_async_copy(row_hbm.at[sid, pl.ds(g * GRP, GRP)],
                                  rowring.at[slot], semr[slot]),
        )

    for src, ch, pc in _PASSES:
        @pl.when(cid == pc)
        def _(src=src, ch=ch):
            pltpu.sync_copy(zer_hbm, acc.at[pl.ds(sid * ROWS_PT, ROWS_PT)])
            plsc.subcore_barrier()

            def gather_desc(j, b, slot, k):
                if src == 0:
                    return pltpu.make_async_copy(
                        hh[ch].at[colring.at[slot, k]], gbuf.at[b], semg[b])
                return pltpu.make_async_copy(
                    r[ch].at[pl.ds(sid * TPE + j * BE, BE)], gbuf.at[b], semg[b])

            dc, dr = idx_desc(0, 0)
            dc.start()
            dr.start()
            dc.wait()
            dr.wait()
            gather_desc(0, 0, 0, 0).start()

            def superblk(gp, _):
                for half in range(2):
                    g = gp * 2 + half

                    @pl.when(g + 1 < NGRP)
                    def _(half=half, g=g):
                        ndc, ndr = idx_desc(g + 1, (half + 1) % 2)
                        ndc.start()
                        ndr.start()
                    for k in range(GRP):
                        j = g * GRP + k
                        gather_desc(j, k % 2, half, k).wait()
                        nk = (k + 1) % GRP
                        nslot = half if k < GRP - 1 else (half + 1) % 2

                        @pl.when(j + 1 < NBLK)
                        def _(j=j, k=k, nk=nk, nslot=nslot, half=half, g=g):
                            if k == GRP - 1:
                                ndc, ndr = idx_desc(g + 1, nslot)
                                ndc.wait()
                                ndr.wait()
                            gather_desc(j + 1, (k + 1) % 2, nslot, nk).start()
                        pltpu.sync_copy(gbuf.at[k % 2],
                                        acc.at[rowring.at[half, k]], add=True)
                return 0
            lax.fori_loop(0, NGRP // 2, superblk, 0)
            plsc.subcore_barrier()
            pltpu.sync_copy(acc.at[pl.ds(sid * ROWS_PT, ROWS_PT)],
                            s_out[src][ch].at[pl.ds(sid * ROWS_PT, ROWS_PT)])


def kernel(x, edge_index, edge_attr, params):
    row = edge_index[0]
    col = edge_index[1]
    h = x.reshape(x.shape[0] * x.shape[1], x.shape[-1])
    h = jnp.pad(h, ((0, NPAD - N), (0, 0)))

    npd = TPE - EPT
    padcol = (jnp.arange(npd, dtype=jnp.int32) * 131) % N
    padrow = N + jnp.arange(npd, dtype=jnp.int32)
    col3d = jnp.concatenate(
        [col.reshape(NS, EPT), jnp.broadcast_to(padcol, (NS, npd))],
        axis=1).reshape(NS, NBLK, BE)
    row3d = jnp.concatenate(
        [row.reshape(NS, EPT), jnp.broadcast_to(padrow, (NS, npd))],
        axis=1).reshape(NS, NBLK, BE)
    ea_pad = jnp.concatenate(
        [edge_attr.reshape(NS, EPT, -1),
         jnp.zeros((NS, TPE - EPT, edge_attr.shape[-1]), jnp.float32)],
        axis=1).reshape(NS * TPE, -1)
    ones_cnt = jnp.ones((BE_CNT, DC), jnp.float32)
    zer_zdc = jnp.zeros((ZROWS, DC), jnp.float32)
    zer_full = jnp.zeros((ROWS_PT, DC), jnp.float32)
    cnt0, cnt1 = _cnt_kernel(row, ones_cnt, zer_zdc)

    for p in params["layers"]:
        hh_c = _node_mlp(h, p["ne_W1"].T, p["ne_b1"].reshape(1, HID),
                         p["ne_W2"].T, p["ne_b2"].reshape(1, HID))
        r_c = _edge_mlp(ea_pad, p["ee_W1"].T, p["ee_b1"].reshape(1, HID))
        s = _segsum_kernel(*hh_c, *r_c, col3d, row3d, zer_full)
        mwt = p["m_W"].T
        c2t = p["ee_W2"].T @ mwt
        dvec = (p["ee_b2"] @ mwt + p["m_b"]).reshape(1, HID)
        h = _fold(s[:NCH], s[NCH:], cnt0, cnt1, mwt, c2t, dvec,
                  p["u_W"].T, p["u_b"].reshape(1, HID))

    out = _out_linear(h, params["out_W"].T, params["out_b"].reshape(1, -1))
    return out[:N].reshape(x.shape[0], x.shape[1], -1)


# speedup vs baseline: 3.1169x; 1.0074x over previous
import functools

import jax
import jax.numpy as jnp
from jax import lax
from jax.experimental import pallas as pl
from jax.experimental.pallas import tpu as pltpu
from jax.experimental.pallas import tpu_sc as plsc

N = 10000
NPAD = 10240
E = 160000
HID = 512
DC = 128
NCH = HID // DC
NS = 16
EPT = E // NS
BE = 128
ROWS_PT = NPAD // NS
ZROWS = 128
BN = 1024
BEDGE = 2048

_sc_mesh = plsc.VectorSubcoreMesh(core_axis_name="c", subcore_axis_name="s")

_PASSES = (
    (0, 0, 0), (0, 1, 0), (1, 0, 0), (1, 1, 0),
    (0, 2, 1), (0, 3, 1), (1, 2, 1), (1, 3, 1),
)


def _mlp2_body(x_ref, w1_ref, b1_ref, w2_ref, b2_ref, *o_refs):
    q = jnp.maximum(
        jnp.dot(x_ref[...].astype(jnp.bfloat16), w1_ref[...],
                preferred_element_type=jnp.float32) + b1_ref[...], 0.0)
    hh = (jnp.dot(q.astype(jnp.bfloat16), w2_ref[...],
                  preferred_element_type=jnp.float32) + b2_ref[...])
    for c in range(NCH):
        o_refs[c][...] = hh[:, c * DC:(c + 1) * DC]


def _node_mlp(h, w1t, b1, w2t, b2):
    din = h.shape[1]
    return pl.pallas_call(
        _mlp2_body,
        grid=(NPAD // BN,),
        in_specs=[
            pl.BlockSpec((BN, din), lambda i: (i, 0)),
            pl.BlockSpec((din, HID), lambda i: (0, 0)),
            pl.BlockSpec((1, HID), lambda i: (0, 0)),
            pl.BlockSpec((HID, HID), lambda i: (0, 0)),
            pl.BlockSpec((1, HID), lambda i: (0, 0)),
        ],
        out_specs=[pl.BlockSpec((BN, DC), lambda i: (i, 0)) for _ in range(NCH)],
        out_shape=[jax.ShapeDtypeStruct((NPAD, DC), jnp.float32) for _ in range(NCH)],
    )(h, w1t, b1, w2t, b2)


def _edge_relu_body(a_ref, w_ref, b_ref, *o_refs):
    q = jnp.maximum(
        jnp.dot(a_ref[...].astype(jnp.bfloat16), w_ref[...],
                preferred_element_type=jnp.float32) + b_ref[...], 0.0)
    for c in range(NCH):
        o_refs[c][...] = q[:, c * DC:(c + 1) * DC]


def _edge_mlp(ea, w1t, b1):
    ed = ea.shape[1]
    ne = ea.shape[0]
    return pl.pallas_call(
        _edge_relu_body,
        grid=(ne // BEDGE,),
        in_specs=[
            pl.BlockSpec((BEDGE, ed), lambda i: (i, 0)),
            pl.BlockSpec((ed, HID), lambda i: (0, 0)),
            pl.BlockSpec((1, HID), lambda i: (0, 0)),
        ],
        out_specs=[pl.BlockSpec((BEDGE, DC), lambda i: (i, 0)) for _ in range(NCH)],
        out_shape=[jax.ShapeDtypeStruct((ne, DC), jnp.float32) for _ in range(NCH)],
    )(ea, w1t, b1)


def _fold_body(s10, s11, s12, s13, s20, s21, s22, s23, c0_ref, c1_ref,
               mwt_ref, c2t_ref, dvec_ref, uwt_ref, ub_ref, o_ref):
    s1 = (s10, s11, s12, s13)
    s2 = (s20, s21, s22, s23)
    cnt = c0_ref[:, 0:1] + c1_ref[:, 0:1]
    sums = cnt * dvec_ref[...]
    for c in range(NCH):
        sums += jnp.dot(s1[c][...].astype(jnp.bfloat16),
                        mwt_ref[c * DC:(c + 1) * DC, :],
                        preferred_element_type=jnp.float32)
        sums += jnp.dot(s2[c][...].astype(jnp.bfloat16),
                        c2t_ref[c * DC:(c + 1) * DC, :],
                        preferred_element_type=jnp.float32)
    inv = 1.0 / jnp.maximum(cnt, 1.0)
    o_ref[...] = (jnp.dot((sums * inv).astype(jnp.bfloat16), uwt_ref[...],
                          preferred_element_type=jnp.float32) + ub_ref[...])


def _fold(s1c, s2c, cnt0, cnt1, mwt, c2t, dvec, uwt, ub):
    chunk_spec = [pl.BlockSpec((BN, DC), lambda i: (i, 0)) for _ in range(2 * NCH)]
    return pl.pallas_call(
        _fold_body,
        grid=(NPAD // BN,),
        in_specs=chunk_spec + [
            pl.BlockSpec((BN, DC), lambda i: (i, 0)),
            pl.BlockSpec((BN, DC), lambda i: (i, 0)),
            pl.BlockSpec((HID, HID), lambda i: (0, 0)),
            pl.BlockSpec((HID, HID), lambda i: (0, 0)),
            pl.BlockSpec((1, HID), lambda i: (0, 0)),
            pl.BlockSpec((HID, HID), lambda i: (0, 0)),
            pl.BlockSpec((1, HID), lambda i: (0, 0)),
        ],
        out_specs=pl.BlockSpec((BN, HID), lambda i: (i, 0)),
        out_shape=jax.ShapeDtypeStruct((NPAD, HID), jnp.float32),
    )(*s1c, *s2c, cnt0, cnt1, mwt, c2t, dvec, uwt, ub)


def _linear_body(x_ref, w_ref, b_ref, o_ref):
    o_ref[...] = (jnp.dot(x_ref[...].astype(jnp.bfloat16), w_ref[...],
                          preferred_element_type=jnp.float32) + b_ref[...])


def _out_linear(h, wt, b):
    dout = wt.shape[1]
    return pl.pallas_call(
        _linear_body,
        grid=(NPAD // BN,),
        in_specs=[
            pl.BlockSpec((BN, HID), lambda i: (i, 0)),
            pl.BlockSpec((HID, dout), lambda i: (0, 0)),
            pl.BlockSpec((1, dout), lambda i: (0, 0)),
        ],
        out_specs=pl.BlockSpec((BN, dout), lambda i: (i, 0)),
        out_shape=jax.ShapeDtypeStruct((NPAD, dout), jnp.float32),
    )(h, wt, b)


@functools.partial(
    pl.kernel,
    out_type=(jax.ShapeDtypeStruct((NPAD, DC), jnp.float32),
              jax.ShapeDtypeStruct((NPAD, DC), jnp.float32)),
    mesh=_sc_mesh,
    scratch_types=[
        pltpu.VMEM((2, 8, BE), jnp.int32),
        pltpu.VMEM((BE, DC), jnp.float32),
        pltpu.VMEM_SHARED((NPAD, DC), jnp.float32),
        pltpu.SemaphoreType.DMA,
        pltpu.SemaphoreType.DMA,
    ],
)
def _cnt_kernel(row_hbm, ones_hbm, zer_hbm, out0, out1, rowring, onesbuf, acc,
                semr0, semr1):
    semr = (semr0, semr1)
    cid = lax.axis_index("c")
    sid = lax.axis_index("s")
    hgrp = 5
    pltpu.sync_copy(ones_hbm, onesbuf)
    pltpu.sync_copy(zer_hbm, acc.at[pl.ds(sid * ROWS_PT, ROWS_PT)])
    plsc.subcore_barrier()

    def idx_desc(g, slot):
        return pltpu.make_async_copy(
            row_hbm.at[sid, pl.ds((cid * hgrp + g) * GRP, GRP)],
            rowring.at[slot], semr[slot])

    idx_desc(0, 0).start()
    for g in range(hgrp):
        slot = g % 2
        idx_desc(g, slot).wait()
        if g + 1 < hgrp:
            idx_desc(g + 1, (g + 1) % 2).start()
        for k in range(GRP):
            pltpu.sync_copy(onesbuf, acc.at[rowring.at[slot, k]], add=True)
    plsc.subcore_barrier()

    @pl.when(cid == 0)
    def _():
        pltpu.sync_copy(acc.at[pl.ds(sid * ROWS_PT, ROWS_PT)],
                        out0.at[pl.ds(sid * ROWS_PT, ROWS_PT)])

    @pl.when(cid == 1)
    def _():
        pltpu.sync_copy(acc.at[pl.ds(sid * ROWS_PT, ROWS_PT)],
                        out1.at[pl.ds(sid * ROWS_PT, ROWS_PT)])


_SEG_OUT = tuple(jax.ShapeDtypeStruct((NPAD, DC), jnp.float32) for _ in range(2 * NCH))


NBUF = 5
TPE = 10240
NBLK = TPE // BE
GRP = 8
NGRP = NBLK // GRP


@functools.partial(
    pl.kernel,
    out_type=_SEG_OUT,
    mesh=_sc_mesh,
    scratch_types=[
        pltpu.VMEM((2, GRP, BE), jnp.int32),
        pltpu.VMEM((2, GRP, BE), jnp.int32),
        pltpu.VMEM((2, BE, DC), jnp.float32),
        pltpu.VMEM_SHARED((NPAD, DC), jnp.float32),
        pltpu.SemaphoreType.DMA,
        pltpu.SemaphoreType.DMA,
        pltpu.SemaphoreType.DMA,
        pltpu.SemaphoreType.DMA,
        pltpu.SemaphoreType.DMA,
        pltpu.SemaphoreType.DMA,
    ],
)
def _segsum_kernel(hh0, hh1, hh2, hh3, r0, r1, r2, r3, col_hbm, row_hbm, zer_hbm,
                   s10, s11, s12, s13, s20, s21, s22, s23,
                   colring, rowring, gbuf, acc,
                   semg0, semg1, semc0, semc1, semr0, semr1):
    hh = (hh0, hh1, hh2, hh3)
    r = (r0, r1, r2, r3)
    s_out = ((s10, s11, s12, s13), (s20, s21, s22, s23))
    semg = (semg0, semg1)
    semc = (semc0, semc1)
    semr = (semr0, semr1)
    cid = lax.axis_index("c")
    sid = lax.axis_index("s")

    def idx_desc(g, slot):
        return (
            pltpu.make_async_copy(col_hbm.at[sid, pl.ds(g * GRP, GRP)],
                                  colring.at[slot], semc[slot]),
            pltpu.make_async_copy(row_hbm.at[sid, pl.ds(g * GRP, GRP)],
                                  rowring.at[slot], semr[slot]),
        )

    for src, ch, pc in _PASSES:
        @pl.when(cid == pc)
        def _(src=src, ch=ch):
            pltpu.sync_copy(zer_hbm, acc.at[pl.ds(sid * ROWS_PT, ROWS_PT)])
            plsc.subcore_barrier()

            def gather_desc(j, b, slot, k):
                if src == 0:
                    return pltpu.make_async_copy(
                        hh[ch].at[colring.at[slot, k]], gbuf.at[b], semg[b])
                return pltpu.make_async_copy(
                    r[ch].at[pl.ds(sid * TPE + j * BE, BE)], gbuf.at[b], semg[b])

            dc, dr = idx_desc(0, 0)
            dc.start()
            dr.start()
            dc.wait()
            dr.wait()
            gather_desc(0, 0, 0, 0).start()

            def superblk(gp, _):
                for half in range(2):
                    g = gp * 2 + half

                    @pl.when(g + 1 < NGRP)
                    def _(half=half, g=g):
                        ndc, ndr = idx_desc(g + 1, (half + 1) % 2)
                        ndc.start()
                        ndr.start()
                    for k in range(GRP):
                        j = g * GRP + k
                        gather_desc(j, k % 2, half, k).wait()
                        nk = (k + 1) % GRP
                        nslot = half if k < GRP - 1 else (half + 1) % 2

                        @pl.when(j + 1 < NBLK)
                        def _(j=j, k=k, nk=nk, nslot=nslot, half=half, g=g):
                            if k == GRP - 1:
                                ndc, ndr = idx_desc(g + 1, nslot)
                                ndc.wait()
                                ndr.wait()
                            gather_desc(j + 1, (k + 1) % 2, nslot, nk).start()
                        pltpu.sync_copy(gbuf.at[k % 2],
                                        acc.at[rowring.at[half, k]], add=True)
                return 0
            lax.fori_loop(0, NGRP // 2, superblk, 0)
            plsc.subcore_barrier()
            pltpu.sync_copy(acc.at[pl.ds(sid * ROWS_PT, ROWS_PT)],
                            s_out[src][ch].at[pl.ds(sid * ROWS_PT, ROWS_PT)])


def kernel(x, edge_index, edge_attr, params):
    row = edge_index[0]
    col = edge_index[1]
    h = x.reshape(x.shape[0] * x.shape[1], x.shape[-1])
    h = jnp.pad(h, ((0, NPAD - N), (0, 0)))

    npd = TPE - EPT
    padcol = (jnp.arange(npd, dtype=jnp.int32) * 131) % N
    padrow = N + jnp.arange(npd, dtype=jnp.int32)
    col3d = jnp.concatenate(
        [col.reshape(NS, EPT), jnp.broadcast_to(padcol, (NS, npd))],
        axis=1).reshape(NS, NBLK, BE)
    row3d = jnp.concatenate(
        [row.reshape(NS, EPT), jnp.broadcast_to(padrow, (NS, npd))],
        axis=1).reshape(NS, NBLK, BE)
    ea_pad = jnp.concatenate(
        [edge_attr.reshape(NS, EPT, -1),
         jnp.zeros((NS, TPE - EPT, edge_attr.shape[-1]), jnp.float32)],
        axis=1).reshape(NS * TPE, -1)
    ones_cnt = jnp.ones((BE, DC), jnp.float32)
    zer_full = jnp.zeros((ROWS_PT, DC), jnp.float32)
    cnt0, cnt1 = _cnt_kernel(row3d, ones_cnt, zer_full)

    for p in params["layers"]:
        bf = jnp.bfloat16
        hh_c = _node_mlp(h, p["ne_W1"].T.astype(bf), p["ne_b1"].reshape(1, HID),
                         p["ne_W2"].T.astype(bf), p["ne_b2"].reshape(1, HID))
        r_c = _edge_mlp(ea_pad, p["ee_W1"].T.astype(bf), p["ee_b1"].reshape(1, HID))
        s = _segsum_kernel(*hh_c, *r_c, col3d, row3d, zer_full)
        mwt = p["m_W"].T
        c2t = p["ee_W2"].T @ mwt
        dvec = (p["ee_b2"] @ mwt + p["m_b"]).reshape(1, HID)
        h = _fold(s[:NCH], s[NCH:], cnt0, cnt1, mwt.astype(bf), c2t.astype(bf),
                  dvec, p["u_W"].T.astype(bf), p["u_b"].reshape(1, HID))

    out = _out_linear(h, params["out_W"].T.astype(jnp.bfloat16),
                      params["out_b"].reshape(1, -1))
    return out[:N].reshape(x.shape[0], x.shape[1], -1)


# speedup vs baseline: 3.6573x; 1.1734x over previous
import functools

import jax
import jax.numpy as jnp
from jax import lax
from jax.experimental import pallas as pl
from jax.experimental.pallas import tpu as pltpu
from jax.experimental.pallas import tpu_sc as plsc

N = 10000
NPAD = 10240
E = 160000
HID = 512
DC = 128
NCH = HID // DC
NS = 16
EPT = E // NS
BE = 80
ROWS_PT = NPAD // NS
ZROWS = 128
BN = 1024
BEDGE = 2048

_sc_mesh = plsc.VectorSubcoreMesh(core_axis_name="c", subcore_axis_name="s")

_PASSES = (
    (0, 0, 0), (0, 1, 0), (1, 0, 0), (1, 1, 0),
    (0, 2, 1), (0, 3, 1), (1, 2, 1), (1, 3, 1),
)


def _mlp2_body(x_ref, w1_ref, b1_ref, w2_ref, b2_ref, *o_refs):
    q = jnp.maximum(
        jnp.dot(x_ref[...].astype(jnp.bfloat16), w1_ref[...],
                preferred_element_type=jnp.float32) + b1_ref[...], 0.0)
    hh = (jnp.dot(q.astype(jnp.bfloat16), w2_ref[...],
                  preferred_element_type=jnp.float32) + b2_ref[...])
    for c in range(NCH):
        o_refs[c][...] = hh[:, c * DC:(c + 1) * DC]


def _node_mlp(h, w1t, b1, w2t, b2):
    din = h.shape[1]
    return pl.pallas_call(
        _mlp2_body,
        grid=(NPAD // BN,),
        in_specs=[
            pl.BlockSpec((BN, din), lambda i: (i, 0)),
            pl.BlockSpec((din, HID), lambda i: (0, 0)),
            pl.BlockSpec((1, HID), lambda i: (0, 0)),
            pl.BlockSpec((HID, HID), lambda i: (0, 0)),
            pl.BlockSpec((1, HID), lambda i: (0, 0)),
        ],
        out_specs=[pl.BlockSpec((BN, DC), lambda i: (i, 0)) for _ in range(NCH)],
        out_shape=[jax.ShapeDtypeStruct((NPAD, DC), jnp.float32) for _ in range(NCH)],
    )(h, w1t, b1, w2t, b2)


def _edge_relu_body(a_ref, w_ref, b_ref, *o_refs):
    q = jnp.maximum(
        jnp.dot(a_ref[...].astype(jnp.bfloat16), w_ref[...],
                preferred_element_type=jnp.float32) + b_ref[...], 0.0)
    for c in range(NCH):
        o_refs[c][...] = q[:, c * DC:(c + 1) * DC]


def _edge_mlp(ea, w1t, b1):
    ed = ea.shape[1]
    ne = ea.shape[0]
    return pl.pallas_call(
        _edge_relu_body,
        grid=(ne // BEDGE,),
        in_specs=[
            pl.BlockSpec((BEDGE, ed), lambda i: (i, 0)),
            pl.BlockSpec((ed, HID), lambda i: (0, 0)),
            pl.BlockSpec((1, HID), lambda i: (0, 0)),
        ],
        out_specs=[pl.BlockSpec((BEDGE, DC), lambda i: (i, 0)) for _ in range(NCH)],
        out_shape=[jax.ShapeDtypeStruct((ne, DC), jnp.float32) for _ in range(NCH)],
    )(ea, w1t, b1)


def _fold_body(s10, s11, s12, s13, s20, s21, s22, s23, c0_ref, c1_ref,
               mwt_ref, c2t_ref, dvec_ref, uwt_ref, ub_ref, o_ref):
    s1 = (s10, s11, s12, s13)
    s2 = (s20, s21, s22, s23)
    cnt = c0_ref[:, 0:1] + c1_ref[:, 0:1]
    sums = cnt * dvec_ref[...]
    for c in range(NCH):
        sums += jnp.dot(s1[c][...].astype(jnp.bfloat16),
                        mwt_ref[c * DC:(c + 1) * DC, :],
                        preferred_element_type=jnp.float32)
        sums += jnp.dot(s2[c][...].astype(jnp.bfloat16),
                        c2t_ref[c * DC:(c + 1) * DC, :],
                        preferred_element_type=jnp.float32)
    inv = 1.0 / jnp.maximum(cnt, 1.0)
    o_ref[...] = (jnp.dot((sums * inv).astype(jnp.bfloat16), uwt_ref[...],
                          preferred_element_type=jnp.float32) + ub_ref[...])


def _fold(s1c, s2c, cnt0, cnt1, mwt, c2t, dvec, uwt, ub):
    chunk_spec = [pl.BlockSpec((BN, DC), lambda i: (i, 0)) for _ in range(2 * NCH)]
    return pl.pallas_call(
        _fold_body,
        grid=(NPAD // BN,),
        in_specs=chunk_spec + [
            pl.BlockSpec((BN, DC), lambda i: (i, 0)),
            pl.BlockSpec((BN, DC), lambda i: (i, 0)),
            pl.BlockSpec((HID, HID), lambda i: (0, 0)),
            pl.BlockSpec((HID, HID), lambda i: (0, 0)),
            pl.BlockSpec((1, HID), lambda i: (0, 0)),
            pl.BlockSpec((HID, HID), lambda i: (0, 0)),
            pl.BlockSpec((1, HID), lambda i: (0, 0)),
        ],
        out_specs=pl.BlockSpec((BN, HID), lambda i: (i, 0)),
        out_shape=jax.ShapeDtypeStruct((NPAD, HID), jnp.float32),
    )(*s1c, *s2c, cnt0, cnt1, mwt, c2t, dvec, uwt, ub)


def _linear_body(x_ref, w_ref, b_ref, o_ref):
    o_ref[...] = (jnp.dot(x_ref[...].astype(jnp.bfloat16), w_ref[...],
                          preferred_element_type=jnp.float32) + b_ref[...])


def _out_linear(h, wt, b):
    dout = wt.shape[1]
    return pl.pallas_call(
        _linear_body,
        grid=(NPAD // BN,),
        in_specs=[
            pl.BlockSpec((BN, HID), lambda i: (i, 0)),
            pl.BlockSpec((HID, dout), lambda i: (0, 0)),
            pl.BlockSpec((1, dout), lambda i: (0, 0)),
        ],
        out_specs=pl.BlockSpec((BN, dout), lambda i: (i, 0)),
        out_shape=jax.ShapeDtypeStruct((NPAD, dout), jnp.float32),
    )(h, wt, b)


@functools.partial(
    pl.kernel,
    out_type=(jax.ShapeDtypeStruct((NPAD, DC), jnp.float32),
              jax.ShapeDtypeStruct((NPAD, DC), jnp.float32)),
    mesh=_sc_mesh,
    scratch_types=[
        pltpu.VMEM((2, 8, BE), jnp.int32),
        pltpu.VMEM((BE, DC), jnp.float32),
        pltpu.VMEM_SHARED((NPAD, DC), jnp.float32),
        pltpu.SemaphoreType.DMA,
        pltpu.SemaphoreType.DMA,
    ],
)
def _cnt_kernel(row_hbm, ones_hbm, zer_hbm, out0, out1, rowring, onesbuf, acc,
                semr0, semr1):
    semr = (semr0, semr1)
    cid = lax.axis_index("c")
    sid = lax.axis_index("s")
    hgrp = NGRP // 2
    pltpu.sync_copy(ones_hbm, onesbuf)
    pltpu.sync_copy(zer_hbm, acc.at[pl.ds(sid * ROWS_PT, ROWS_PT)])
    plsc.subcore_barrier()

    def idx_desc(g, slot):
        return pltpu.make_async_copy(
            row_hbm.at[sid, pl.ds((cid * hgrp + g) * GRP, GRP)],
            rowring.at[slot], semr[slot])

    idx_desc(0, 0).start()
    for g in range(hgrp):
        slot = g % 2
        idx_desc(g, slot).wait()
        if g + 1 < hgrp:
            idx_desc(g + 1, (g + 1) % 2).start()
        for k in range(GRP):
            pltpu.sync_copy(onesbuf, acc.at[rowring.at[slot, k]], add=True)
    plsc.subcore_barrier()

    @pl.when(cid == 0)
    def _():
        pltpu.sync_copy(acc.at[pl.ds(sid * ROWS_PT, ROWS_PT)],
                        out0.at[pl.ds(sid * ROWS_PT, ROWS_PT)])

    @pl.when(cid == 1)
    def _():
        pltpu.sync_copy(acc.at[pl.ds(sid * ROWS_PT, ROWS_PT)],
                        out1.at[pl.ds(sid * ROWS_PT, ROWS_PT)])


_SEG_OUT = tuple(jax.ShapeDtypeStruct((NPAD, DC), jnp.float32) for _ in range(2 * NCH))


NBUF = 5
TPE = 10240
NBLK = TPE // BE
GRP = 8
NGRP = NBLK // GRP


NBUF = 4
LOOK = NBUF - 1


@functools.partial(
    pl.kernel,
    out_type=_SEG_OUT,
    mesh=_sc_mesh,
    scratch_types=[
        pltpu.VMEM((2, GRP, BE), jnp.int32),
        pltpu.VMEM((2, GRP, BE), jnp.int32),
        pltpu.VMEM((NBUF, BE, DC), jnp.float32),
        pltpu.VMEM_SHARED((NPAD, DC), jnp.float32),
    ] + [pltpu.SemaphoreType.DMA] * (NBUF + 4),
)
def _segsum_kernel(hh0, hh1, hh2, hh3, r0, r1, r2, r3, col_hbm, row_hbm, zer_hbm,
                   s10, s11, s12, s13, s20, s21, s22, s23,
                   colring, rowring, gbuf, acc, *sems):
    hh = (hh0, hh1, hh2, hh3)
    r = (r0, r1, r2, r3)
    s_out = ((s10, s11, s12, s13), (s20, s21, s22, s23))
    semg = sems[:NBUF]
    semc = sems[NBUF:NBUF + 2]
    semr = sems[NBUF + 2:NBUF + 4]
    cid = lax.axis_index("c")
    sid = lax.axis_index("s")

    def idx_desc(g, slot):
        return (
            pltpu.make_async_copy(col_hbm.at[sid, pl.ds(g * GRP, GRP)],
                                  colring.at[slot], semc[slot]),
            pltpu.make_async_copy(row_hbm.at[sid, pl.ds(g * GRP, GRP)],
                                  rowring.at[slot], semr[slot]),
        )

    for src, ch, pc in _PASSES:
        @pl.when(cid == pc)
        def _(src=src, ch=ch):
            pltpu.sync_copy(zer_hbm, acc.at[pl.ds(sid * ROWS_PT, ROWS_PT)])
            plsc.subcore_barrier()

            def gather_desc(j, b, slot, k):
                if src == 0:
                    return pltpu.make_async_copy(
                        hh[ch].at[colring.at[slot, k]], gbuf.at[b], semg[b])
                return pltpu.make_async_copy(
                    r[ch].at[pl.ds(sid * TPE + j * BE, BE)], gbuf.at[b], semg[b])

            dc, dr = idx_desc(0, 0)
            dc.start()
            dr.start()
            dc.wait()
            dr.wait()
            for b in range(LOOK):
                gather_desc(b, b, 0, b).start()

            def superblk(gp, _):
                for half in range(2):
                    g = gp * 2 + half

                    @pl.when(g + 1 < NGRP)
                    def _(half=half, g=g):
                        ndc, ndr = idx_desc(g + 1, (half + 1) % 2)
                        ndc.start()
                        ndr.start()
                    for k in range(GRP):
                        j = g * GRP + k
                        gather_desc(j, k % NBUF, half, k).wait()
                        nk = (k + LOOK) % GRP
                        nslot = half if k < GRP - LOOK else (half + 1) % 2

                        @pl.when(j + LOOK < NBLK)
                        def _(j=j, k=k, nk=nk, nslot=nslot, half=half, g=g):
                            if k == GRP - LOOK:
                                ndc, ndr = idx_desc(g + 1, nslot)
                                ndc.wait()
                                ndr.wait()
                            gather_desc(j + LOOK, (k + LOOK) % NBUF, nslot, nk).start()
                        pltpu.sync_copy(gbuf.at[k % NBUF],
                                        acc.at[rowring.at[half, k]], add=True)
                return 0
            lax.fori_loop(0, NGRP // 2, superblk, 0)
            plsc.subcore_barrier()
            pltpu.sync_copy(acc.at[pl.ds(sid * ROWS_PT, ROWS_PT)],
                            s_out[src][ch].at[pl.ds(sid * ROWS_PT, ROWS_PT)])


def kernel(x, edge_index, edge_attr, params):
    row = edge_index[0]
    col = edge_index[1]
    h = x.reshape(x.shape[0] * x.shape[1], x.shape[-1])
    h = jnp.pad(h, ((0, NPAD - N), (0, 0)))

    npd = TPE - EPT
    padcol = (jnp.arange(npd, dtype=jnp.int32) * 131) % N
    padrow = N + jnp.arange(npd, dtype=jnp.int32)
    col3d = jnp.concatenate(
        [col.reshape(NS, EPT), jnp.broadcast_to(padcol, (NS, npd))],
        axis=1).reshape(NS, NBLK, BE)
    row3d = jnp.concatenate(
        [row.reshape(NS, EPT), jnp.broadcast_to(padrow, (NS, npd))],
        axis=1).reshape(NS, NBLK, BE)
    ea_pad = jnp.concatenate(
        [edge_attr.reshape(NS, EPT, -1),
         jnp.zeros((NS, TPE - EPT, edge_attr.shape[-1]), jnp.float32)],
        axis=1).reshape(NS * TPE, -1)
    ones_cnt = jnp.ones((BE, DC), jnp.float32)
    zer_full = jnp.zeros((ROWS_PT, DC), jnp.float32)
    cnt0, cnt1 = _cnt_kernel(row3d, ones_cnt, zer_full)

    for p in params["layers"]:
        bf = jnp.bfloat16
        hh_c = _node_mlp(h, p["ne_W1"].T.astype(bf), p["ne_b1"].reshape(1, HID),
                         p["ne_W2"].T.astype(bf), p["ne_b2"].reshape(1, HID))
        r_c = _edge_mlp(ea_pad, p["ee_W1"].T.astype(bf), p["ee_b1"].reshape(1, HID))
        s = _segsum_kernel(*hh_c, *r_c, col3d, row3d, zer_full)
        mwt = p["m_W"].T
        c2t = p["ee_W2"].T @ mwt
        dvec = (p["ee_b2"] @ mwt + p["m_b"]).reshape(1, HID)
        h = _fold(s[:NCH], s[NCH:], cnt0, cnt1, mwt.astype(bf), c2t.astype(bf),
                  dvec, p["u_W"].T.astype(bf), p["u_b"].reshape(1, HID))

    out = _out_linear(h, params["out_W"].T.astype(jnp.bfloat16),
                      params["out_b"].reshape(1, -1))
    return out[:N].reshape(x.shape[0], x.shape[1], -1)


# speedup vs baseline: 3.7299x; 1.0199x over previous
import functools

import jax
import jax.numpy as jnp
from jax import lax
from jax.experimental import pallas as pl
from jax.experimental.pallas import tpu as pltpu
from jax.experimental.pallas import tpu_sc as plsc

N = 10000
NPAD = 10240
E = 160000
HID = 512
DC = 128
NCH = HID // DC
NS = 16
EPT = E // NS
BE = 80
ROWS_PT = NPAD // NS
ZROWS = 128
BN = 1024
BEDGE = 2048

_sc_mesh = plsc.VectorSubcoreMesh(core_axis_name="c", subcore_axis_name="s")

_PASSES = (
    (0, 0, 0), (0, 1, 0), (1, 0, 0), (1, 1, 0),
    (0, 2, 1), (0, 3, 1), (1, 2, 1), (1, 3, 1),
)


def _mlp2_body(x_ref, w1_ref, b1_ref, w2_ref, b2_ref, *o_refs):
    q = jnp.maximum(
        jnp.dot(x_ref[...].astype(jnp.bfloat16), w1_ref[...],
                preferred_element_type=jnp.float32) + b1_ref[...], 0.0)
    hh = (jnp.dot(q.astype(jnp.bfloat16), w2_ref[...],
                  preferred_element_type=jnp.float32) + b2_ref[...])
    for c in range(NCH):
        o_refs[c][...] = hh[:, c * DC:(c + 1) * DC]


def _node_mlp(h, w1t, b1, w2t, b2):
    din = h.shape[1]
    return pl.pallas_call(
        _mlp2_body,
        grid=(NPAD // BN,),
        in_specs=[
            pl.BlockSpec((BN, din), lambda i: (i, 0)),
            pl.BlockSpec((din, HID), lambda i: (0, 0)),
            pl.BlockSpec((1, HID), lambda i: (0, 0)),
            pl.BlockSpec((HID, HID), lambda i: (0, 0)),
            pl.BlockSpec((1, HID), lambda i: (0, 0)),
        ],
        out_specs=[pl.BlockSpec((BN, DC), lambda i: (i, 0)) for _ in range(NCH)],
        out_shape=[jax.ShapeDtypeStruct((NPAD, DC), jnp.float32) for _ in range(NCH)],
    )(h, w1t, b1, w2t, b2)


def _edge_relu_body(a_ref, w_ref, b_ref, *o_refs):
    q = jnp.maximum(
        jnp.dot(a_ref[...].astype(jnp.bfloat16), w_ref[...],
                preferred_element_type=jnp.float32) + b_ref[...], 0.0)
    for c in range(NCH):
        o_refs[c][...] = q[:, c * DC:(c + 1) * DC]


def _edge_mlp(ea, w1t, b1):
    ed = ea.shape[1]
    ne = ea.shape[0]
    return pl.pallas_call(
        _edge_relu_body,
        grid=(ne // BEDGE,),
        in_specs=[
            pl.BlockSpec((BEDGE, ed), lambda i: (i, 0)),
            pl.BlockSpec((ed, HID), lambda i: (0, 0)),
            pl.BlockSpec((1, HID), lambda i: (0, 0)),
        ],
        out_specs=[pl.BlockSpec((BEDGE, DC), lambda i: (i, 0)) for _ in range(NCH)],
        out_shape=[jax.ShapeDtypeStruct((ne, DC), jnp.float32) for _ in range(NCH)],
    )(ea, w1t, b1)


def _fold_body(s10, s11, s12, s13, s20, s21, s22, s23, c0_ref, c1_ref,
               mwt_ref, c2t_ref, dvec_ref, uwt_ref, ub_ref, o_ref):
    s1 = (s10, s11, s12, s13)
    s2 = (s20, s21, s22, s23)
    cnt = c0_ref[:, 0:1] + c1_ref[:, 0:1]
    sums = cnt * dvec_ref[...]
    for c in range(NCH):
        sums += jnp.dot(s1[c][...].astype(jnp.bfloat16),
                        mwt_ref[c * DC:(c + 1) * DC, :],
                        preferred_element_type=jnp.float32)
        sums += jnp.dot(s2[c][...].astype(jnp.bfloat16),
                        c2t_ref[c * DC:(c + 1) * DC, :],
                        preferred_element_type=jnp.float32)
    inv = 1.0 / jnp.maximum(cnt, 1.0)
    o_ref[...] = (jnp.dot((sums * inv).astype(jnp.bfloat16), uwt_ref[...],
                          preferred_element_type=jnp.float32) + ub_ref[...])


def _fold(s1c, s2c, cnt0, cnt1, mwt, c2t, dvec, uwt, ub):
    chunk_spec = [pl.BlockSpec((BN, DC), lambda i: (i, 0)) for _ in range(2 * NCH)]
    return pl.pallas_call(
        _fold_body,
        grid=(NPAD // BN,),
        in_specs=chunk_spec + [
            pl.BlockSpec((BN, DC), lambda i: (i, 0)),
            pl.BlockSpec((BN, DC), lambda i: (i, 0)),
            pl.BlockSpec((HID, HID), lambda i: (0, 0)),
            pl.BlockSpec((HID, HID), lambda i: (0, 0)),
            pl.BlockSpec((1, HID), lambda i: (0, 0)),
            pl.BlockSpec((HID, HID), lambda i: (0, 0)),
            pl.BlockSpec((1, HID), lambda i: (0, 0)),
        ],
        out_specs=pl.BlockSpec((BN, HID), lambda i: (i, 0)),
        out_shape=jax.ShapeDtypeStruct((NPAD, HID), jnp.float32),
    )(*s1c, *s2c, cnt0, cnt1, mwt, c2t, dvec, uwt, ub)


def _fold_part(s1, s2, c0_ref, c1_ref, mwt_ref, c2t_ref, dvec_ref,
               uwt_ref, ub_ref):
    cnt = c0_ref[:, 0:1] + c1_ref[:, 0:1]
    sums = cnt * dvec_ref[...]
    for c in range(NCH):
        sums += jnp.dot(s1[c][...].astype(jnp.bfloat16),
                        mwt_ref[c * DC:(c + 1) * DC, :],
                        preferred_element_type=jnp.float32)
        sums += jnp.dot(s2[c][...].astype(jnp.bfloat16),
                        c2t_ref[c * DC:(c + 1) * DC, :],
                        preferred_element_type=jnp.float32)
    inv = 1.0 / jnp.maximum(cnt, 1.0)
    return (jnp.dot((sums * inv).astype(jnp.bfloat16), uwt_ref[...],
                    preferred_element_type=jnp.float32) + ub_ref[...])


def _fold_mlp_body(s10, s11, s12, s13, s20, s21, s22, s23, c0_ref, c1_ref,
                   mwt_ref, c2t_ref, dvec_ref, uwt_ref, ub_ref,
                   w1_ref, b1_ref, w2_ref, b2_ref, *o_refs):
    h = _fold_part((s10, s11, s12, s13), (s20, s21, s22, s23), c0_ref, c1_ref,
                   mwt_ref, c2t_ref, dvec_ref, uwt_ref, ub_ref)
    q = jnp.maximum(
        jnp.dot(h.astype(jnp.bfloat16), w1_ref[...],
                preferred_element_type=jnp.float32) + b1_ref[...], 0.0)
    hh = (jnp.dot(q.astype(jnp.bfloat16), w2_ref[...],
                  preferred_element_type=jnp.float32) + b2_ref[...])
    for c in range(NCH):
        o_refs[c][...] = hh[:, c * DC:(c + 1) * DC]


def _fold_mlp(s1c, s2c, cnt0, cnt1, mwt, c2t, dvec, uwt, ub, w1t, b1, w2t, b2):
    chunk_spec = [pl.BlockSpec((BN, DC), lambda i: (i, 0)) for _ in range(2 * NCH)]
    wspec = pl.BlockSpec((HID, HID), lambda i: (0, 0))
    bspec = pl.BlockSpec((1, HID), lambda i: (0, 0))
    return pl.pallas_call(
        _fold_mlp_body,
        grid=(NPAD // BN,),
        in_specs=chunk_spec + [
            pl.BlockSpec((BN, DC), lambda i: (i, 0)),
            pl.BlockSpec((BN, DC), lambda i: (i, 0)),
            wspec, wspec, bspec, wspec, bspec, wspec, bspec, wspec, bspec,
        ],
        out_specs=[pl.BlockSpec((BN, DC), lambda i: (i, 0)) for _ in range(NCH)],
        out_shape=[jax.ShapeDtypeStruct((NPAD, DC), jnp.float32) for _ in range(NCH)],
    )(*s1c, *s2c, cnt0, cnt1, mwt, c2t, dvec, uwt, ub, w1t, b1, w2t, b2)


def _fold_out_body(s10, s11, s12, s13, s20, s21, s22, s23, c0_ref, c1_ref,
                   mwt_ref, c2t_ref, dvec_ref, uwt_ref, ub_ref,
                   wo_ref, bo_ref, o_ref):
    h = _fold_part((s10, s11, s12, s13), (s20, s21, s22, s23), c0_ref, c1_ref,
                   mwt_ref, c2t_ref, dvec_ref, uwt_ref, ub_ref)
    o_ref[...] = (jnp.dot(h.astype(jnp.bfloat16), wo_ref[...],
                          preferred_element_type=jnp.float32) + bo_ref[...])


def _fold_out(s1c, s2c, cnt0, cnt1, mwt, c2t, dvec, uwt, ub, wo, bo):
    dout = wo.shape[1]
    chunk_spec = [pl.BlockSpec((BN, DC), lambda i: (i, 0)) for _ in range(2 * NCH)]
    wspec = pl.BlockSpec((HID, HID), lambda i: (0, 0))
    bspec = pl.BlockSpec((1, HID), lambda i: (0, 0))
    return pl.pallas_call(
        _fold_out_body,
        grid=(NPAD // BN,),
        in_specs=chunk_spec + [
            pl.BlockSpec((BN, DC), lambda i: (i, 0)),
            pl.BlockSpec((BN, DC), lambda i: (i, 0)),
            wspec, wspec, bspec, wspec, bspec,
            pl.BlockSpec((HID, dout), lambda i: (0, 0)),
            pl.BlockSpec((1, dout), lambda i: (0, 0)),
        ],
        out_specs=pl.BlockSpec((BN, dout), lambda i: (i, 0)),
        out_shape=jax.ShapeDtypeStruct((NPAD, dout), jnp.float32),
    )(*s1c, *s2c, cnt0, cnt1, mwt, c2t, dvec, uwt, ub, wo, bo)


def _linear_body(x_ref, w_ref, b_ref, o_ref):
    o_ref[...] = (jnp.dot(x_ref[...].astype(jnp.bfloat16), w_ref[...],
                          preferred_element_type=jnp.float32) + b_ref[...])


def _out_linear(h, wt, b):
    dout = wt.shape[1]
    return pl.pallas_call(
        _linear_body,
        grid=(NPAD // BN,),
        in_specs=[
            pl.BlockSpec((BN, HID), lambda i: (i, 0)),
            pl.BlockSpec((HID, dout), lambda i: (0, 0)),
            pl.BlockSpec((1, dout), lambda i: (0, 0)),
        ],
        out_specs=pl.BlockSpec((BN, dout), lambda i: (i, 0)),
        out_shape=jax.ShapeDtypeStruct((NPAD, dout), jnp.float32),
    )(h, wt, b)


@functools.partial(
    pl.kernel,
    out_type=(jax.ShapeDtypeStruct((NPAD, DC), jnp.float32),
              jax.ShapeDtypeStruct((NPAD, DC), jnp.float32)),
    mesh=_sc_mesh,
    scratch_types=[
        pltpu.VMEM((2, 8, BE), jnp.int32),
        pltpu.VMEM((BE, DC), jnp.float32),
        pltpu.VMEM_SHARED((NPAD, DC), jnp.float32),
        pltpu.SemaphoreType.DMA,
        pltpu.SemaphoreType.DMA,
    ],
)
def _cnt_kernel(row_hbm, ones_hbm, zer_hbm, out0, out1, rowring, onesbuf, acc,
                semr0, semr1):
    semr = (semr0, semr1)
    cid = lax.axis_index("c")
    sid = lax.axis_index("s")
    hgrp = NGRP // 2
    pltpu.sync_copy(ones_hbm, onesbuf)
    pltpu.sync_copy(zer_hbm, acc.at[pl.ds(sid * ROWS_PT, ROWS_PT)])
    plsc.subcore_barrier()

    def idx_desc(g, slot):
        return pltpu.make_async_copy(
            row_hbm.at[sid, pl.ds((cid * hgrp + g) * GRP, GRP)],
            rowring.at[slot], semr[slot])

    idx_desc(0, 0).start()
    for g in range(hgrp):
        slot = g % 2
        idx_desc(g, slot).wait()
        if g + 1 < hgrp:
            idx_desc(g + 1, (g + 1) % 2).start()
        for k in range(GRP):
            pltpu.sync_copy(onesbuf, acc.at[rowring.at[slot, k]], add=True)
    plsc.subcore_barrier()

    @pl.when(cid == 0)
    def _():
        pltpu.sync_copy(acc.at[pl.ds(sid * ROWS_PT, ROWS_PT)],
                        out0.at[pl.ds(sid * ROWS_PT, ROWS_PT)])

    @pl.when(cid == 1)
    def _():
        pltpu.sync_copy(acc.at[pl.ds(sid * ROWS_PT, ROWS_PT)],
                        out1.at[pl.ds(sid * ROWS_PT, ROWS_PT)])


_SEG_OUT = tuple(jax.ShapeDtypeStruct((NPAD, DC), jnp.float32) for _ in range(2 * NCH))


NBUF = 5
TPE = 10240
NBLK = TPE // BE
GRP = 8
NGRP = NBLK // GRP


NBUF = 4
LOOK = NBUF - 1


@functools.partial(
    pl.kernel,
    out_type=_SEG_OUT,
    mesh=_sc_mesh,
    scratch_types=[
        pltpu.VMEM((2, GRP, BE), jnp.int32),
        pltpu.VMEM((2, GRP, BE), jnp.int32),
        pltpu.VMEM((NBUF, BE, DC), jnp.float32),
        pltpu.VMEM_SHARED((NPAD, DC), jnp.float32),
    ] + [pltpu.SemaphoreType.DMA] * (NBUF + 4),
)
def _segsum_kernel(hh0, hh1, hh2, hh3, r0, r1, r2, r3, col_hbm, row_hbm, zer_hbm,
                   s10, s11, s12, s13, s20, s21, s22, s23,
                   colring, rowring, gbuf, acc, *sems):
    hh = (hh0, hh1, hh2, hh3)
    r = (r0, r1, r2, r3)
    s_out = ((s10, s11, s12, s13), (s20, s21, s22, s23))
    semg = sems[:NBUF]
    semc = sems[NBUF:NBUF + 2]
    semr = sems[NBUF + 2:NBUF + 4]
    cid = lax.axis_index("c")
    sid = lax.axis_index("s")

    def idx_desc(g, slot):
        return (
            pltpu.make_async_copy(col_hbm.at[sid, pl.ds(g * GRP, GRP)],
                                  colring.at[slot], semc[slot]),
            pltpu.make_async_copy(row_hbm.at[sid, pl.ds(g * GRP, GRP)],
                                  rowring.at[slot], semr[slot]),
        )

    for src, ch, pc in _PASSES:
        @pl.when(cid == pc)
        def _(src=src, ch=ch):
            pltpu.sync_copy(zer_hbm, acc.at[pl.ds(sid * ROWS_PT, ROWS_PT)])
            plsc.subcore_barrier()

            def gather_desc(j, b, slot, k):
                if src == 0:
                    return pltpu.make_async_copy(
                        hh[ch].at[colring.at[slot, k]], gbuf.at[b], semg[b])
                return pltpu.make_async_copy(
                    r[ch].at[pl.ds(sid * TPE + j * BE, BE)], gbuf.at[b], semg[b])

            dc, dr = idx_desc(0, 0)
            dc.start()
            dr.start()
            dc.wait()
            dr.wait()
            for b in range(LOOK):
                gather_desc(b, b, 0, b).start()

            def superblk(gp, _):
                for half in range(2):
                    g = gp * 2 + half

                    @pl.when(g + 1 < NGRP)
                    def _(half=half, g=g):
                        ndc, ndr = idx_desc(g + 1, (half + 1) % 2)
                        ndc.start()
                        ndr.start()
                    for k in range(GRP):
                        j = g * GRP + k
                        gather_desc(j, k % NBUF, half, k).wait()
                        nk = (k + LOOK) % GRP
                        nslot = half if k < GRP - LOOK else (half + 1) % 2

                        @pl.when(j + LOOK < NBLK)
                        def _(j=j, k=k, nk=nk, nslot=nslot, half=half, g=g):
                            if k == GRP - LOOK:
                                ndc, ndr = idx_desc(g + 1, nslot)
                                ndc.wait()
                                ndr.wait()
                            gather_desc(j + LOOK, (k + LOOK) % NBUF, nslot, nk).start()
                        pltpu.sync_copy(gbuf.at[k % NBUF],
                                        acc.at[rowring.at[half, k]], add=True)
                return 0
            lax.fori_loop(0, NGRP // 2, superblk, 0)
            plsc.subcore_barrier()
            pltpu.sync_copy(acc.at[pl.ds(sid * ROWS_PT, ROWS_PT)],
                            s_out[src][ch].at[pl.ds(sid * ROWS_PT, ROWS_PT)])


def kernel(x, edge_index, edge_attr, params):
    row = edge_index[0]
    col = edge_index[1]
    h = x.reshape(x.shape[0] * x.shape[1], x.shape[-1])
    h = jnp.pad(h, ((0, NPAD - N), (0, 0)))

    npd = TPE - EPT
    padcol = (jnp.arange(npd, dtype=jnp.int32) * 131) % N
    padrow = N + jnp.arange(npd, dtype=jnp.int32)
    col3d = jnp.concatenate(
        [col.reshape(NS, EPT), jnp.broadcast_to(padcol, (NS, npd))],
        axis=1).reshape(NS, NBLK, BE)
    row3d = jnp.concatenate(
        [row.reshape(NS, EPT), jnp.broadcast_to(padrow, (NS, npd))],
        axis=1).reshape(NS, NBLK, BE)
    ea_pad = jnp.concatenate(
        [edge_attr.reshape(NS, EPT, -1),
         jnp.zeros((NS, TPE - EPT, edge_attr.shape[-1]), jnp.float32)],
        axis=1).reshape(NS * TPE, -1)
    ones_cnt = jnp.ones((BE, DC), jnp.float32)
    zer_full = jnp.zeros((ROWS_PT, DC), jnp.float32)
    cnt0, cnt1 = _cnt_kernel(row3d, ones_cnt, zer_full)

    bf = jnp.bfloat16
    layers = params["layers"]
    p0 = layers[0]
    hh_c = _node_mlp(h, p0["ne_W1"].T.astype(bf), p0["ne_b1"].reshape(1, HID),
                     p0["ne_W2"].T.astype(bf), p0["ne_b2"].reshape(1, HID))
    for li, p in enumerate(layers):
        r_c = _edge_mlp(ea_pad, p["ee_W1"].T.astype(bf), p["ee_b1"].reshape(1, HID))
        s = _segsum_kernel(*hh_c, *r_c, col3d, row3d, zer_full)
        mwt = p["m_W"].T
        c2t = p["ee_W2"].T @ mwt
        dvec = (p["ee_b2"] @ mwt + p["m_b"]).reshape(1, HID)
        fold_args = (s[:NCH], s[NCH:], cnt0, cnt1, mwt.astype(bf),
                     c2t.astype(bf), dvec, p["u_W"].T.astype(bf),
                     p["u_b"].reshape(1, HID))
        if li + 1 < len(layers):
            pn = layers[li + 1]
            hh_c = _fold_mlp(*fold_args,
                             pn["ne_W1"].T.astype(bf), pn["ne_b1"].reshape(1, HID),
                             pn["ne_W2"].T.astype(bf), pn["ne_b2"].reshape(1, HID))
        else:
            out = _fold_out(*fold_args, params["out_W"].T.astype(bf),
                            params["out_b"].reshape(1, -1))
    return out[:N].reshape(x.shape[0], x.shape[1], -1)


# speedup vs baseline: 3.7665x; 1.0098x over previous
import functools

import jax
import jax.numpy as jnp
from jax import lax
from jax.experimental import pallas as pl
from jax.experimental.pallas import tpu as pltpu
from jax.experimental.pallas import tpu_sc as plsc

N = 10000
NPAD = 10240
E = 160000
HID = 512
DC = 128
NCH = HID // DC
NS = 16
EPT = E // NS
BE = 80
ROWS_PT = NPAD // NS
ZROWS = 128
BN = 1024
BEDGE = 2048

_sc_mesh = plsc.VectorSubcoreMesh(core_axis_name="c", subcore_axis_name="s")

_PASSES = (
    (0, 0, 0), (0, 1, 0), (1, 0, 0), (1, 1, 0),
    (0, 2, 1), (0, 3, 1), (1, 2, 1), (1, 3, 1),
)


def _mlp2_body(x_ref, w1_ref, b1_ref, w2_ref, b2_ref, *o_refs):
    q = jnp.maximum(
        jnp.dot(x_ref[...].astype(jnp.bfloat16), w1_ref[...],
                preferred_element_type=jnp.float32) + b1_ref[...], 0.0)
    hh = (jnp.dot(q.astype(jnp.bfloat16), w2_ref[...],
                  preferred_element_type=jnp.float32) + b2_ref[...])
    for c in range(NCH):
        o_refs[c][...] = hh[:, c * DC:(c + 1) * DC]


def _node_mlp(h, w1t, b1, w2t, b2):
    din = h.shape[1]
    return pl.pallas_call(
        _mlp2_body,
        grid=(NPAD // BN,),
        in_specs=[
            pl.BlockSpec((BN, din), lambda i: (i, 0)),
            pl.BlockSpec((din, HID), lambda i: (0, 0)),
            pl.BlockSpec((1, HID), lambda i: (0, 0)),
            pl.BlockSpec((HID, HID), lambda i: (0, 0)),
            pl.BlockSpec((1, HID), lambda i: (0, 0)),
        ],
        out_specs=[pl.BlockSpec((BN, DC), lambda i: (i, 0)) for _ in range(NCH)],
        out_shape=[jax.ShapeDtypeStruct((NPAD, DC), jnp.float32) for _ in range(NCH)],
    )(h, w1t, b1, w2t, b2)


def _edge_relu_body(a_ref, w_ref, b_ref, *o_refs):
    q = jnp.maximum(
        jnp.dot(a_ref[...].astype(jnp.bfloat16), w_ref[...],
                preferred_element_type=jnp.float32) + b_ref[...], 0.0)
    for c in range(NCH):
        o_refs[c][...] = q[:, c * DC:(c + 1) * DC]


def _edge_mlp(ea, w1t, b1):
    ed = ea.shape[1]
    ne = ea.shape[0]
    return pl.pallas_call(
        _edge_relu_body,
        grid=(ne // BEDGE,),
        in_specs=[
            pl.BlockSpec((BEDGE, ed), lambda i: (i, 0)),
            pl.BlockSpec((ed, HID), lambda i: (0, 0)),
            pl.BlockSpec((1, HID), lambda i: (0, 0)),
        ],
        out_specs=[pl.BlockSpec((BEDGE, DC), lambda i: (i, 0)) for _ in range(NCH)],
        out_shape=[jax.ShapeDtypeStruct((ne, DC), jnp.float32) for _ in range(NCH)],
    )(ea, w1t, b1)


def _fold_body(s10, s11, s12, s13, s20, s21, s22, s23, c0_ref, c1_ref,
               mwt_ref, c2t_ref, dvec_ref, uwt_ref, ub_ref, o_ref):
    s1 = (s10, s11, s12, s13)
    s2 = (s20, s21, s22, s23)
    cnt = c0_ref[:, 0:1] + c1_ref[:, 0:1]
    sums = cnt * dvec_ref[...]
    for c in range(NCH):
        sums += jnp.dot(s1[c][...].astype(jnp.bfloat16),
                        mwt_ref[c * DC:(c + 1) * DC, :],
                        preferred_element_type=jnp.float32)
        sums += jnp.dot(s2[c][...].astype(jnp.bfloat16),
                        c2t_ref[c * DC:(c + 1) * DC, :],
                        preferred_element_type=jnp.float32)
    inv = 1.0 / jnp.maximum(cnt, 1.0)
    o_ref[...] = (jnp.dot((sums * inv).astype(jnp.bfloat16), uwt_ref[...],
                          preferred_element_type=jnp.float32) + ub_ref[...])


def _fold(s1c, s2c, cnt0, cnt1, mwt, c2t, dvec, uwt, ub):
    chunk_spec = [pl.BlockSpec((BN, DC), lambda i: (i, 0)) for _ in range(2 * NCH)]
    return pl.pallas_call(
        _fold_body,
        grid=(NPAD // BN,),
        in_specs=chunk_spec + [
            pl.BlockSpec((BN, DC), lambda i: (i, 0)),
            pl.BlockSpec((BN, DC), lambda i: (i, 0)),
            pl.BlockSpec((HID, HID), lambda i: (0, 0)),
            pl.BlockSpec((HID, HID), lambda i: (0, 0)),
            pl.BlockSpec((1, HID), lambda i: (0, 0)),
            pl.BlockSpec((HID, HID), lambda i: (0, 0)),
            pl.BlockSpec((1, HID), lambda i: (0, 0)),
        ],
        out_specs=pl.BlockSpec((BN, HID), lambda i: (i, 0)),
        out_shape=jax.ShapeDtypeStruct((NPAD, HID), jnp.float32),
    )(*s1c, *s2c, cnt0, cnt1, mwt, c2t, dvec, uwt, ub)


def _fold_part(s1, s2, c0_ref, c1_ref, mwt_ref, c2t_ref, dvec_ref,
               uwt_ref, ub_ref):
    cnt = c0_ref[:, 0:1] + c1_ref[:, 0:1]
    sums = cnt * dvec_ref[...]
    for c in range(NCH):
        sums += jnp.dot(s1[c][...].astype(jnp.bfloat16),
                        mwt_ref[c * DC:(c + 1) * DC, :],
                        preferred_element_type=jnp.float32)
        sums += jnp.dot(s2[c][...].astype(jnp.bfloat16),
                        c2t_ref[c * DC:(c + 1) * DC, :],
                        preferred_element_type=jnp.float32)
    inv = 1.0 / jnp.maximum(cnt, 1.0)
    return (jnp.dot((sums * inv).astype(jnp.bfloat16), uwt_ref[...],
                    preferred_element_type=jnp.float32) + ub_ref[...])


def _fold_mlp_body(s10, s11, s12, s13, s20, s21, s22, s23, c0_ref, c1_ref,
                   mwt_ref, c2t_ref, dvec_ref, uwt_ref, ub_ref,
                   w1_ref, b1_ref, w2_ref, b2_ref, *o_refs):
    h = _fold_part((s10, s11, s12, s13), (s20, s21, s22, s23), c0_ref, c1_ref,
                   mwt_ref, c2t_ref, dvec_ref, uwt_ref, ub_ref)
    q = jnp.maximum(
        jnp.dot(h.astype(jnp.bfloat16), w1_ref[...],
                preferred_element_type=jnp.float32) + b1_ref[...], 0.0)
    hh = (jnp.dot(q.astype(jnp.bfloat16), w2_ref[...],
                  preferred_element_type=jnp.float32) + b2_ref[...])
    for c in range(NCH):
        o_refs[c][...] = hh[:, c * DC:(c + 1) * DC]


def _fold_mlp(s1c, s2c, cnt0, cnt1, mwt, c2t, dvec, uwt, ub, w1t, b1, w2t, b2):
    chunk_spec = [pl.BlockSpec((BN, DC), lambda i: (i, 0)) for _ in range(2 * NCH)]
    wspec = pl.BlockSpec((HID, HID), lambda i: (0, 0))
    bspec = pl.BlockSpec((1, HID), lambda i: (0, 0))
    return pl.pallas_call(
        _fold_mlp_body,
        grid=(NPAD // BN,),
        in_specs=chunk_spec + [
            pl.BlockSpec((BN, DC), lambda i: (i, 0)),
            pl.BlockSpec((BN, DC), lambda i: (i, 0)),
            wspec, wspec, bspec, wspec, bspec, wspec, bspec, wspec, bspec,
        ],
        out_specs=[pl.BlockSpec((BN, DC), lambda i: (i, 0)) for _ in range(NCH)],
        out_shape=[jax.ShapeDtypeStruct((NPAD, DC), jnp.float32) for _ in range(NCH)],
    )(*s1c, *s2c, cnt0, cnt1, mwt, c2t, dvec, uwt, ub, w1t, b1, w2t, b2)


def _fold_out_body(s10, s11, s12, s13, s20, s21, s22, s23, c0_ref, c1_ref,
                   mwt_ref, c2t_ref, dvec_ref, uwt_ref, ub_ref,
                   wo_ref, bo_ref, o_ref):
    h = _fold_part((s10, s11, s12, s13), (s20, s21, s22, s23), c0_ref, c1_ref,
                   mwt_ref, c2t_ref, dvec_ref, uwt_ref, ub_ref)
    o_ref[...] = (jnp.dot(h.astype(jnp.bfloat16), wo_ref[...],
                          preferred_element_type=jnp.float32) + bo_ref[...])


def _fold_out(s1c, s2c, cnt0, cnt1, mwt, c2t, dvec, uwt, ub, wo, bo):
    dout = wo.shape[1]
    chunk_spec = [pl.BlockSpec((BN, DC), lambda i: (i, 0)) for _ in range(2 * NCH)]
    wspec = pl.BlockSpec((HID, HID), lambda i: (0, 0))
    bspec = pl.BlockSpec((1, HID), lambda i: (0, 0))
    return pl.pallas_call(
        _fold_out_body,
        grid=(NPAD // BN,),
        in_specs=chunk_spec + [
            pl.BlockSpec((BN, DC), lambda i: (i, 0)),
            pl.BlockSpec((BN, DC), lambda i: (i, 0)),
            wspec, wspec, bspec, wspec, bspec,
            pl.BlockSpec((HID, dout), lambda i: (0, 0)),
            pl.BlockSpec((1, dout), lambda i: (0, 0)),
        ],
        out_specs=pl.BlockSpec((BN, dout), lambda i: (i, 0)),
        out_shape=jax.ShapeDtypeStruct((NPAD, dout), jnp.float32),
    )(*s1c, *s2c, cnt0, cnt1, mwt, c2t, dvec, uwt, ub, wo, bo)


def _linear_body(x_ref, w_ref, b_ref, o_ref):
    o_ref[...] = (jnp.dot(x_ref[...].astype(jnp.bfloat16), w_ref[...],
                          preferred_element_type=jnp.float32) + b_ref[...])


def _out_linear(h, wt, b):
    dout = wt.shape[1]
    return pl.pallas_call(
        _linear_body,
        grid=(NPAD // BN,),
        in_specs=[
            pl.BlockSpec((BN, HID), lambda i: (i, 0)),
            pl.BlockSpec((HID, dout), lambda i: (0, 0)),
            pl.BlockSpec((1, dout), lambda i: (0, 0)),
        ],
        out_specs=pl.BlockSpec((BN, dout), lambda i: (i, 0)),
        out_shape=jax.ShapeDtypeStruct((NPAD, dout), jnp.float32),
    )(h, wt, b)


@functools.partial(
    pl.kernel,
    out_type=(jax.ShapeDtypeStruct((NPAD, DC), jnp.float32),
              jax.ShapeDtypeStruct((NPAD, DC), jnp.float32)),
    mesh=_sc_mesh,
    scratch_types=[
        pltpu.VMEM((2, 8, BE), jnp.int32),
        pltpu.VMEM((BE, DC), jnp.float32),
        pltpu.VMEM_SHARED((NPAD, DC), jnp.float32),
        pltpu.SemaphoreType.DMA,
        pltpu.SemaphoreType.DMA,
    ],
)
def _cnt_kernel(row_hbm, ones_hbm, zer_hbm, out0, out1, rowring, onesbuf, acc,
                semr0, semr1):
    semr = (semr0, semr1)
    cid = lax.axis_index("c")
    sid = lax.axis_index("s")
    hgrp = NGRP // 2
    pltpu.sync_copy(ones_hbm, onesbuf)
    pltpu.sync_copy(zer_hbm, acc.at[pl.ds(sid * ROWS_PT, ROWS_PT)])
    plsc.subcore_barrier()

    def idx_desc(g, slot):
        return pltpu.make_async_copy(
            row_hbm.at[sid, pl.ds((cid * hgrp + g) * GRP, GRP)],
            rowring.at[slot], semr[slot])

    idx_desc(0, 0).start()
    for g in range(hgrp):
        slot = g % 2
        idx_desc(g, slot).wait()
        if g + 1 < hgrp:
            idx_desc(g + 1, (g + 1) % 2).start()
        for k in range(GRP):
            pltpu.sync_copy(onesbuf, acc.at[rowring.at[slot, k]], add=True)
    plsc.subcore_barrier()

    @pl.when(cid == 0)
    def _():
        pltpu.sync_copy(acc.at[pl.ds(sid * ROWS_PT, ROWS_PT)],
                        out0.at[pl.ds(sid * ROWS_PT, ROWS_PT)])

    @pl.when(cid == 1)
    def _():
        pltpu.sync_copy(acc.at[pl.ds(sid * ROWS_PT, ROWS_PT)],
                        out1.at[pl.ds(sid * ROWS_PT, ROWS_PT)])


_SEG_OUT = tuple(jax.ShapeDtypeStruct((NPAD, DC), jnp.float32) for _ in range(2 * NCH))


NBUF = 5
TPE = 10240
NBLK = TPE // BE
GRP = 8
NGRP = NBLK // GRP


NBUF = 4
LOOK = NBUF - 1


@functools.partial(
    pl.kernel,
    out_type=_SEG_OUT,
    mesh=_sc_mesh,
    scratch_types=[
        pltpu.VMEM((2, GRP, BE), jnp.int32),
        pltpu.VMEM((2, GRP, BE), jnp.int32),
        pltpu.VMEM((NBUF, BE, DC), jnp.float32),
        pltpu.VMEM_SHARED((NPAD, DC), jnp.float32),
    ] + [pltpu.SemaphoreType.DMA] * (NBUF + 5),
)
def _segsum_kernel(hh0, hh1, hh2, hh3, r0, r1, r2, r3, col_hbm, row_hbm, zer_hbm,
                   s10, s11, s12, s13, s20, s21, s22, s23,
                   colring, rowring, gbuf, acc, *sems):
    hh = (hh0, hh1, hh2, hh3)
    r = (r0, r1, r2, r3)
    s_out = ((s10, s11, s12, s13), (s20, s21, s22, s23))
    semg = sems[:NBUF]
    semc = sems[NBUF:NBUF + 2]
    semr = sems[NBUF + 2:NBUF + 4]
    semz = sems[NBUF + 4]
    cid = lax.axis_index("c")
    sid = lax.axis_index("s")

    def idx_desc(g, slot):
        return (
            pltpu.make_async_copy(col_hbm.at[sid, pl.ds(g * GRP, GRP)],
                                  colring.at[slot], semc[slot]),
            pltpu.make_async_copy(row_hbm.at[sid, pl.ds(g * GRP, GRP)],
                                  rowring.at[slot], semr[slot]),
        )

    for src, ch, pc in _PASSES:
        @pl.when(cid == pc)
        def _(src=src, ch=ch):
            zd = pltpu.make_async_copy(
                zer_hbm, acc.at[pl.ds(sid * ROWS_PT, ROWS_PT)], semz)
            zd.start()

            def gather_desc(j, b, slot, k):
                if src == 0:
                    return pltpu.make_async_copy(
                        hh[ch].at[colring.at[slot, k]], gbuf.at[b], semg[b])
                return pltpu.make_async_copy(
                    r[ch].at[pl.ds(sid * TPE + j * BE, BE)], gbuf.at[b], semg[b])

            dc, dr = idx_desc(0, 0)
            dc.start()
            dr.start()
            dc.wait()
            dr.wait()
            for b in range(LOOK):
                gather_desc(b, b, 0, b).start()
            zd.wait()
            plsc.subcore_barrier()

            def superblk(gp, _):
                for half in range(2):
                    g = gp * 2 + half

                    @pl.when(g + 1 < NGRP)
                    def _(half=half, g=g):
                        ndc, ndr = idx_desc(g + 1, (half + 1) % 2)
                        ndc.start()
                        ndr.start()
                    for k in range(GRP):
                        j = g * GRP + k
                        gather_desc(j, k % NBUF, half, k).wait()
                        nk = (k + LOOK) % GRP
                        nslot = half if k < GRP - LOOK else (half + 1) % 2

                        @pl.when(j + LOOK < NBLK)
                        def _(j=j, k=k, nk=nk, nslot=nslot, half=half, g=g):
                            if k == GRP - LOOK:
                                ndc, ndr = idx_desc(g + 1, nslot)
                                ndc.wait()
                                ndr.wait()
                            gather_desc(j + LOOK, (k + LOOK) % NBUF, nslot, nk).start()
                        pltpu.sync_copy(gbuf.at[k % NBUF],
                                        acc.at[rowring.at[half, k]], add=True)
                return 0
            lax.fori_loop(0, NGRP // 2, superblk, 0)
            plsc.subcore_barrier()
            pltpu.sync_copy(acc.at[pl.ds(sid * ROWS_PT, ROWS_PT)],
                            s_out[src][ch].at[pl.ds(sid * ROWS_PT, ROWS_PT)])


def kernel(x, edge_index, edge_attr, params):
    row = edge_index[0]
    col = edge_index[1]
    h = x.reshape(x.shape[0] * x.shape[1], x.shape[-1])
    h = jnp.pad(h, ((0, NPAD - N), (0, 0)))

    npd = TPE - EPT
    padcol = (jnp.arange(npd, dtype=jnp.int32) * 131) % N
    padrow = N + jnp.arange(npd, dtype=jnp.int32)
    col3d = jnp.concatenate(
        [col.reshape(NS, EPT), jnp.broadcast_to(padcol, (NS, npd))],
        axis=1).reshape(NS, NBLK, BE)
    row3d = jnp.concatenate(
        [row.reshape(NS, EPT), jnp.broadcast_to(padrow, (NS, npd))],
        axis=1).reshape(NS, NBLK, BE)
    ea_pad = jnp.concatenate(
        [edge_attr.reshape(NS, EPT, -1),
         jnp.zeros((NS, TPE - EPT, edge_attr.shape[-1]), jnp.float32)],
        axis=1).reshape(NS * TPE, -1)
    ones_cnt = jnp.ones((BE, DC), jnp.float32)
    zer_full = jnp.zeros((ROWS_PT, DC), jnp.float32)
    cnt0, cnt1 = _cnt_kernel(row3d, ones_cnt, zer_full)

    bf = jnp.bfloat16
    layers = params["layers"]
    p0 = layers[0]
    hh_c = _node_mlp(h, p0["ne_W1"].T.astype(bf), p0["ne_b1"].reshape(1, HID),
                     p0["ne_W2"].T.astype(bf), p0["ne_b2"].reshape(1, HID))
    for li, p in enumerate(layers):
        r_c = _edge_mlp(ea_pad, p["ee_W1"].T.astype(bf), p["ee_b1"].reshape(1, HID))
        s = _segsum_kernel(*hh_c, *r_c, col3d, row3d, zer_full)
        mwt = p["m_W"].T
        c2t = p["ee_W2"].T @ mwt
        dvec = (p["ee_b2"] @ mwt + p["m_b"]).reshape(1, HID)
        fold_args = (s[:NCH], s[NCH:], cnt0, cnt1, mwt.astype(bf),
                     c2t.astype(bf), dvec, p["u_W"].T.astype(bf),
                     p["u_b"].reshape(1, HID))
        if li + 1 < len(layers):
            pn = layers[li + 1]
            hh_c = _fold_mlp(*fold_args,
                             pn["ne_W1"].T.astype(bf), pn["ne_b1"].reshape(1, HID),
                             pn["ne_W2"].T.astype(bf), pn["ne_b2"].reshape(1, HID))
        else:
            out = _fold_out(*fold_args, params["out_W"].T.astype(bf),
                            params["out_b"].reshape(1, -1))
    return out[:N].reshape(x.shape[0], x.shape[1], -1)


# speedup vs baseline: 3.7695x; 1.0008x over previous
import functools

import jax
import jax.numpy as jnp
from jax import lax
from jax.experimental import pallas as pl
from jax.experimental.pallas import tpu as pltpu
from jax.experimental.pallas import tpu_sc as plsc

N = 10000
NPAD = 10240
E = 160000
HID = 512
DC = 128
NCH = HID // DC
NS = 16
EPT = E // NS
BE = 80
ROWS_PT = NPAD // NS
BN = 1024
BEDGE = 2048

_sc_mesh = plsc.VectorSubcoreMesh(core_axis_name="c", subcore_axis_name="s")

_PASSES = (
    (0, 0, 0), (0, 1, 0), (1, 0, 0), (1, 1, 0),
    (0, 2, 1), (0, 3, 1), (1, 2, 1), (1, 3, 1),
)


def _mlp2_body(x_ref, w1_ref, b1_ref, w2_ref, b2_ref, *o_refs):
    q = jnp.maximum(
        jnp.dot(x_ref[...].astype(jnp.bfloat16), w1_ref[...],
                preferred_element_type=jnp.float32) + b1_ref[...], 0.0)
    hh = (jnp.dot(q.astype(jnp.bfloat16), w2_ref[...],
                  preferred_element_type=jnp.float32) + b2_ref[...])
    for c in range(NCH):
        o_refs[c][...] = hh[:, c * DC:(c + 1) * DC]


def _node_mlp(h, w1t, b1, w2t, b2):
    din = h.shape[1]
    return pl.pallas_call(
        _mlp2_body,
        grid=(NPAD // BN,),
        in_specs=[
            pl.BlockSpec((BN, din), lambda i: (i, 0)),
            pl.BlockSpec((din, HID), lambda i: (0, 0)),
            pl.BlockSpec((1, HID), lambda i: (0, 0)),
            pl.BlockSpec((HID, HID), lambda i: (0, 0)),
            pl.BlockSpec((1, HID), lambda i: (0, 0)),
        ],
        out_specs=[pl.BlockSpec((BN, DC), lambda i: (i, 0)) for _ in range(NCH)],
        out_shape=[jax.ShapeDtypeStruct((NPAD, DC), jnp.float32) for _ in range(NCH)],
    )(h, w1t, b1, w2t, b2)


def _edge_relu_body(a_ref, w_ref, b_ref, *o_refs):
    q = jnp.maximum(
        jnp.dot(a_ref[...].astype(jnp.bfloat16), w_ref[...],
                preferred_element_type=jnp.float32) + b_ref[...], 0.0)
    for c in range(NCH):
        o_refs[c][...] = q[:, c * DC:(c + 1) * DC]


def _edge_mlp(ea, w1t, b1):
    ed = ea.shape[1]
    ne = ea.shape[0]
    return pl.pallas_call(
        _edge_relu_body,
        grid=(ne // BEDGE,),
        in_specs=[
            pl.BlockSpec((BEDGE, ed), lambda i: (i, 0)),
            pl.BlockSpec((ed, HID), lambda i: (0, 0)),
            pl.BlockSpec((1, HID), lambda i: (0, 0)),
        ],
        out_specs=[pl.BlockSpec((BEDGE, DC), lambda i: (i, 0)) for _ in range(NCH)],
        out_shape=[jax.ShapeDtypeStruct((ne, DC), jnp.float32) for _ in range(NCH)],
    )(ea, w1t, b1)


def _fold_part(s1, s2, c0_ref, c1_ref, mwt_ref, c2t_ref, dvec_ref,
               uwt_ref, ub_ref):
    cnt = c0_ref[:, 0:1] + c1_ref[:, 0:1]
    sums = cnt * dvec_ref[...]
    for c in range(NCH):
        sums += jnp.dot(s1[c][...].astype(jnp.bfloat16),
                        mwt_ref[c * DC:(c + 1) * DC, :],
                        preferred_element_type=jnp.float32)
        sums += jnp.dot(s2[c][...].astype(jnp.bfloat16),
                        c2t_ref[c * DC:(c + 1) * DC, :],
                        preferred_element_type=jnp.float32)
    inv = 1.0 / jnp.maximum(cnt, 1.0)
    return (jnp.dot((sums * inv).astype(jnp.bfloat16), uwt_ref[...],
                    preferred_element_type=jnp.float32) + ub_ref[...])


def _fold_mlp_body(s10, s11, s12, s13, s20, s21, s22, s23, c0_ref, c1_ref,
                   mwt_ref, c2t_ref, dvec_ref, uwt_ref, ub_ref,
                   w1_ref, b1_ref, w2_ref, b2_ref, *o_refs):
    h = _fold_part((s10, s11, s12, s13), (s20, s21, s22, s23), c0_ref, c1_ref,
                   mwt_ref, c2t_ref, dvec_ref, uwt_ref, ub_ref)
    q = jnp.maximum(
        jnp.dot(h.astype(jnp.bfloat16), w1_ref[...],
                preferred_element_type=jnp.float32) + b1_ref[...], 0.0)
    hh = (jnp.dot(q.astype(jnp.bfloat16), w2_ref[...],
                  preferred_element_type=jnp.float32) + b2_ref[...])
    for c in range(NCH):
        o_refs[c][...] = hh[:, c * DC:(c + 1) * DC]


def _fold_mlp(s1c, s2c, cnt0, cnt1, mwt, c2t, dvec, uwt, ub, w1t, b1, w2t, b2):
    chunk_spec = [pl.BlockSpec((BN, DC), lambda i: (i, 0)) for _ in range(2 * NCH)]
    wspec = pl.BlockSpec((HID, HID), lambda i: (0, 0))
    bspec = pl.BlockSpec((1, HID), lambda i: (0, 0))
    return pl.pallas_call(
        _fold_mlp_body,
        grid=(NPAD // BN,),
        in_specs=chunk_spec + [
            pl.BlockSpec((BN, DC), lambda i: (i, 0)),
            pl.BlockSpec((BN, DC), lambda i: (i, 0)),
            wspec, wspec, bspec, wspec, bspec, wspec, bspec, wspec, bspec,
        ],
        out_specs=[pl.BlockSpec((BN, DC), lambda i: (i, 0)) for _ in range(NCH)],
        out_shape=[jax.ShapeDtypeStruct((NPAD, DC), jnp.float32) for _ in range(NCH)],
    )(*s1c, *s2c, cnt0, cnt1, mwt, c2t, dvec, uwt, ub, w1t, b1, w2t, b2)


def _fold_out_body(s10, s11, s12, s13, s20, s21, s22, s23, c0_ref, c1_ref,
                   mwt_ref, c2t_ref, dvec_ref, uwt_ref, ub_ref,
                   wo_ref, bo_ref, o_ref):
    h = _fold_part((s10, s11, s12, s13), (s20, s21, s22, s23), c0_ref, c1_ref,
                   mwt_ref, c2t_ref, dvec_ref, uwt_ref, ub_ref)
    o_ref[...] = (jnp.dot(h.astype(jnp.bfloat16), wo_ref[...],
                          preferred_element_type=jnp.float32) + bo_ref[...])


def _fold_out(s1c, s2c, cnt0, cnt1, mwt, c2t, dvec, uwt, ub, wo, bo):
    dout = wo.shape[1]
    chunk_spec = [pl.BlockSpec((BN, DC), lambda i: (i, 0)) for _ in range(2 * NCH)]
    wspec = pl.BlockSpec((HID, HID), lambda i: (0, 0))
    bspec = pl.BlockSpec((1, HID), lambda i: (0, 0))
    return pl.pallas_call(
        _fold_out_body,
        grid=(NPAD // BN,),
        in_specs=chunk_spec + [
            pl.BlockSpec((BN, DC), lambda i: (i, 0)),
            pl.BlockSpec((BN, DC), lambda i: (i, 0)),
            wspec, wspec, bspec, wspec, bspec,
            pl.BlockSpec((HID, dout), lambda i: (0, 0)),
            pl.BlockSpec((1, dout), lambda i: (0, 0)),
        ],
        out_specs=pl.BlockSpec((BN, dout), lambda i: (i, 0)),
        out_shape=jax.ShapeDtypeStruct((NPAD, dout), jnp.float32),
    )(*s1c, *s2c, cnt0, cnt1, mwt, c2t, dvec, uwt, ub, wo, bo)


@functools.partial(
    pl.kernel,
    out_type=(jax.ShapeDtypeStruct((NPAD, DC), jnp.float32),
              jax.ShapeDtypeStruct((NPAD, DC), jnp.float32)),
    mesh=_sc_mesh,
    scratch_types=[
        pltpu.VMEM((2, 8, BE), jnp.int32),
        pltpu.VMEM((BE, DC), jnp.float32),
        pltpu.VMEM_SHARED((NPAD, DC), jnp.float32),
        pltpu.SemaphoreType.DMA,
        pltpu.SemaphoreType.DMA,
    ],
)
def _cnt_kernel(row_hbm, ones_hbm, zer_hbm, out0, out1, rowring, onesbuf, acc,
                semr0, semr1):
    semr = (semr0, semr1)
    cid = lax.axis_index("c")
    sid = lax.axis_index("s")
    hgrp = NGRP // 2
    pltpu.sync_copy(ones_hbm, onesbuf)
    pltpu.sync_copy(zer_hbm, acc.at[pl.ds(sid * ROWS_PT, ROWS_PT)])
    plsc.subcore_barrier()

    def idx_desc(g, slot):
        return pltpu.make_async_copy(
            row_hbm.at[sid, pl.ds((cid * hgrp + g) * GRP, GRP)],
            rowring.at[slot], semr[slot])

    idx_desc(0, 0).start()
    for g in range(hgrp):
        slot = g % 2
        idx_desc(g, slot).wait()
        if g + 1 < hgrp:
            idx_desc(g + 1, (g + 1) % 2).start()
        for k in range(GRP):
            pltpu.sync_copy(onesbuf, acc.at[rowring.at[slot, k]], add=True)
    plsc.subcore_barrier()

    @pl.when(cid == 0)
    def _():
        pltpu.sync_copy(acc.at[pl.ds(sid * ROWS_PT, ROWS_PT)],
                        out0.at[pl.ds(sid * ROWS_PT, ROWS_PT)])

    @pl.when(cid == 1)
    def _():
        pltpu.sync_copy(acc.at[pl.ds(sid * ROWS_PT, ROWS_PT)],
                        out1.at[pl.ds(sid * ROWS_PT, ROWS_PT)])


_SEG_OUT = tuple(jax.ShapeDtypeStruct((NPAD, DC), jnp.float32) for _ in range(2 * NCH))


NBUF = 5
TPE = 10240
NBLK = TPE // BE
GRP = 8
NGRP = NBLK // GRP


NBUF = 4
LOOK = NBUF - 1


@functools.partial(
    pl.kernel,
    out_type=_SEG_OUT,
    mesh=_sc_mesh,
    scratch_types=[
        pltpu.VMEM((2, GRP, BE), jnp.int32),
        pltpu.VMEM((2, GRP, BE), jnp.int32),
        pltpu.VMEM((NBUF, BE, DC), jnp.float32),
        pltpu.VMEM_SHARED((NPAD, DC), jnp.float32),
    ] + [pltpu.SemaphoreType.DMA] * (NBUF + 5),
)
def _segsum_kernel(hh0, hh1, hh2, hh3, r0, r1, r2, r3, col_hbm, row_hbm, zer_hbm,
                   s10, s11, s12, s13, s20, s21, s22, s23,
                   colring, rowring, gbuf, acc, *sems):
    hh = (hh0, hh1, hh2, hh3)
    r = (r0, r1, r2, r3)
    s_out = ((s10, s11, s12, s13), (s20, s21, s22, s23))
    semg = sems[:NBUF]
    semc = sems[NBUF:NBUF + 2]
    semr = sems[NBUF + 2:NBUF + 4]
    semz = sems[NBUF + 4]
    cid = lax.axis_index("c")
    sid = lax.axis_index("s")

    def idx_desc(g, slot):
        return (
            pltpu.make_async_copy(col_hbm.at[sid, pl.ds(g * GRP, GRP)],
                                  colring.at[slot], semc[slot]),
            pltpu.make_async_copy(row_hbm.at[sid, pl.ds(g * GRP, GRP)],
                                  rowring.at[slot], semr[slot]),
        )

    for src, ch, pc in _PASSES:
        @pl.when(cid == pc)
        def _(src=src, ch=ch):
            zd = pltpu.make_async_copy(
                zer_hbm, acc.at[pl.ds(sid * ROWS_PT, ROWS_PT)], semz)
            zd.start()

            def gather_desc(j, b, slot, k):
                if src == 0:
                    return pltpu.make_async_copy(
                        hh[ch].at[colring.at[slot, k]], gbuf.at[b], semg[b])
                return pltpu.make_async_copy(
                    r[ch].at[pl.ds(sid * TPE + j * BE, BE)], gbuf.at[b], semg[b])

            dc, dr = idx_desc(0, 0)
            dc.start()
            dr.start()
            dc.wait()
            dr.wait()
            for b in range(LOOK):
                gather_desc(b, b, 0, b).start()
            zd.wait()
            plsc.subcore_barrier()

            def superblk(gp, _):
                for half in range(2):
                    g = gp * 2 + half

                    @pl.when(g + 1 < NGRP)
                    def _(half=half, g=g):
                        ndc, ndr = idx_desc(g + 1, (half + 1) % 2)
                        ndc.start()
                        ndr.start()
                    for k in range(GRP):
                        j = g * GRP + k
                        gather_desc(j, k % NBUF, half, k).wait()
                        nk = (k + LOOK) % GRP
                        nslot = half if k < GRP - LOOK else (half + 1) % 2

                        @pl.when(j + LOOK < NBLK)
                        def _(j=j, k=k, nk=nk, nslot=nslot, half=half, g=g):
                            if k == GRP - LOOK:
                                ndc, ndr = idx_desc(g + 1, nslot)
                                ndc.wait()
                                ndr.wait()
                            gather_desc(j + LOOK, (k + LOOK) % NBUF, nslot, nk).start()
                        pltpu.sync_copy(gbuf.at[k % NBUF],
                                        acc.at[rowring.at[half, k]], add=True)
                return 0
            lax.fori_loop(0, NGRP // 2, superblk, 0)
            plsc.subcore_barrier()
            pltpu.sync_copy(acc.at[pl.ds(sid * ROWS_PT, ROWS_PT)],
                            s_out[src][ch].at[pl.ds(sid * ROWS_PT, ROWS_PT)])


def kernel(x, edge_index, edge_attr, params):
    row = edge_index[0]
    col = edge_index[1]
    h = x.reshape(x.shape[0] * x.shape[1], x.shape[-1])
    h = jnp.pad(h, ((0, NPAD - N), (0, 0)))

    npd = TPE - EPT
    padcol = (jnp.arange(npd, dtype=jnp.int32) * 131) % N
    padrow = N + jnp.arange(npd, dtype=jnp.int32)
    col3d = jnp.concatenate(
        [col.reshape(NS, EPT), jnp.broadcast_to(padcol, (NS, npd))],
        axis=1).reshape(NS, NBLK, BE)
    row3d = jnp.concatenate(
        [row.reshape(NS, EPT), jnp.broadcast_to(padrow, (NS, npd))],
        axis=1).reshape(NS, NBLK, BE)
    ea_pad = jnp.concatenate(
        [edge_attr.reshape(NS, EPT, -1),
         jnp.zeros((NS, TPE - EPT, edge_attr.shape[-1]), jnp.float32)],
        axis=1).reshape(NS * TPE, -1)
    ones_cnt = jnp.ones((BE, DC), jnp.float32)
    zer_full = jnp.zeros((ROWS_PT, DC), jnp.float32)
    cnt0, cnt1 = _cnt_kernel(row3d, ones_cnt, zer_full)

    bf = jnp.bfloat16
    layers = params["layers"]
    p0 = layers[0]
    hh_c = _node_mlp(h, p0["ne_W1"].T.astype(bf), p0["ne_b1"].reshape(1, HID),
                     p0["ne_W2"].T.astype(bf), p0["ne_b2"].reshape(1, HID))
    for li, p in enumerate(layers):
        r_c = _edge_mlp(ea_pad, p["ee_W1"].T.astype(bf), p["ee_b1"].reshape(1, HID))
        s = _segsum_kernel(*hh_c, *r_c, col3d, row3d, zer_full)
        mwt = p["m_W"].T
        c2t = p["ee_W2"].T @ mwt
        dvec = (p["ee_b2"] @ mwt + p["m_b"]).reshape(1, HID)
        fold_args = (s[:NCH], s[NCH:], cnt0, cnt1, mwt.astype(bf),
                     c2t.astype(bf), dvec, p["u_W"].T.astype(bf),
                     p["u_b"].reshape(1, HID))
        if li + 1 < len(layers):
            pn = layers[li + 1]
            hh_c = _fold_mlp(*fold_args,
                             pn["ne_W1"].T.astype(bf), pn["ne_b1"].reshape(1, HID),
                             pn["ne_W2"].T.astype(bf), pn["ne_b2"].reshape(1, HID))
        else:
            out = _fold_out(*fold_args, params["out_W"].T.astype(bf),
                            params["out_b"].reshape(1, -1))
    return out[:N].reshape(x.shape[0], x.shape[1], -1)
